# Initial kernel scaffold; baseline (speedup 1.0000x reference)
import jax
import jax.numpy as jnp
from jax.experimental import pallas as pl


def kernel(x, edge_index, edge_attr, W1, b1, W2, b2, lW1, lb1, lW2, lb2):
    raise NotImplementedError("write your pallas kernel here")


# speedup vs baseline: 4.7072x; 4.7072x over previous
import jax
import jax.numpy as jnp
from jax import lax
from jax.experimental import pallas as pl
from jax.experimental.pallas import tpu as pltpu
from jax.experimental.pallas import tpu_sc as plsc

N = 10000
E = 320000
D = 128
H = 128
DE = 16
C = 2

NC = 2
NS = 16
NW = NC * NS
EPW = E // NW
CHUNK = 80
NCHUNK = EPW // CHUNK
N16 = 10112
RPS = N16 // NS

_MESH = plsc.VectorSubcoreMesh(
    core_axis_name="c", subcore_axis_name="s", num_cores=NC, num_subcores=NS)
_SC_PARAMS = pltpu.CompilerParams(needs_layout_passes=False)


def _wid():
    return lax.axis_index("s") * NC + lax.axis_index("c")


def _deg_body(dst_hbm, hist_hbm, dst_v, hist_v):
    w = _wid()
    pltpu.sync_copy(dst_hbm.at[pl.ds(w * EPW, EPW)], dst_v)
    zeros = jnp.zeros((16,), jnp.float32)

    def _z(i, _):
        hist_v[pl.ds(i * 16, 16)] = zeros
        return 0
    lax.fori_loop(0, N // 16, _z, 0, unroll=8)

    ones = jnp.full((16,), 1.0, jnp.float32)

    def _acc(i, _):
        idx = dst_v[pl.ds(i * 16, 16)]
        plsc.addupdate_scatter(hist_v, [idx], ones)
        return 0
    lax.fori_loop(0, EPW // 16, _acc, 0, unroll=8)
    pltpu.sync_copy(hist_v, hist_hbm.at[pl.ds(w * N, N)])


_deg_call = pl.kernel(
    _deg_body,
    out_type=jax.ShapeDtypeStruct((NW * N,), jnp.float32),
    mesh=_MESH,
    compiler_params=_SC_PARAMS,
    scratch_types=[
        pltpu.VMEM((EPW,), jnp.int32),
        pltpu.VMEM((N,), jnp.float32),
    ],
)


def _seg_body(y_hbm, src_hbm, dst_hbm, out_hbm,
              src_v, dst_v, rows0, rows1, zbuf, acc, sem0, sem1):
    w = _wid()
    sid = lax.axis_index("s")
    cid = lax.axis_index("c")
    pltpu.sync_copy(src_hbm.at[pl.ds(w * EPW, EPW)], src_v)
    pltpu.sync_copy(dst_hbm.at[w], dst_v)

    zeros = jnp.zeros((16,), jnp.float32)

    def _z(i, _):
        zbuf[i // 8, pl.ds((i % 8) * 16, 16)] = zeros
        return 0
    lax.fori_loop(0, 64, _z, 0, unroll=8)

    def _zc(k, _):
        pltpu.sync_copy(zbuf, acc.at[pl.ds(sid * RPS + k * 8, 8)])
        return 0
    lax.fori_loop(0, RPS // 8, _zc, 0)
    plsc.subcore_barrier()

    def _sl(j):
        return src_v.at[pl.ds(j * CHUNK, CHUNK)]

    pltpu.async_copy(y_hbm.at[_sl(0)], rows0, sem0)

    def _step(j, _):
        even = j % 2 == 0

        @pl.when(j + 1 < NCHUNK)
        def _():
            @pl.when(even)
            def _():
                pltpu.async_copy(y_hbm.at[_sl(j + 1)], rows1, sem1)

            @pl.when(jnp.logical_not(even))
            def _():
                pltpu.async_copy(y_hbm.at[_sl(j + 1)], rows0, sem0)

        @pl.when(even)
        def _():
            pltpu.make_async_copy(y_hbm.at[_sl(j)], rows0, sem0).wait()
            pltpu.sync_copy(rows0, acc.at[dst_v.at[j]], add=True)

        @pl.when(jnp.logical_not(even))
        def _():
            pltpu.make_async_copy(y_hbm.at[_sl(j)], rows1, sem1).wait()
            pltpu.sync_copy(rows1, acc.at[dst_v.at[j]], add=True)
        return 0

    lax.fori_loop(0, NCHUNK, _step, 0)
    plsc.subcore_barrier()
    pltpu.sync_copy(acc.at[pl.ds(sid * RPS, RPS)],
                    out_hbm.at[pl.ds(cid * N16 + sid * RPS, RPS)])


_seg_call = pl.kernel(
    _seg_body,
    out_type=jax.ShapeDtypeStruct((NC * N16, D), jnp.float32),
    mesh=_MESH,
    compiler_params=_SC_PARAMS,
    scratch_types=[
        pltpu.VMEM((EPW,), jnp.int32),
        pltpu.VMEM((NCHUNK, CHUNK), jnp.int32),
        pltpu.VMEM((CHUNK, D), jnp.float32),
        pltpu.VMEM((CHUNK, D), jnp.float32),
        pltpu.VMEM((8, D), jnp.float32),
        pltpu.VMEM_SHARED((N16, D), jnp.float32),
        pltpu.SemaphoreType.DMA,
        pltpu.SemaphoreType.DMA,
    ],
)


def _edge_body(a_hbm, b_hbm, ea_hbm, src_hbm, dst_hbm, z_hbm,
               src_v, dst_v, bufa, bufb, bufe, sema, semb, seme):
    w = _wid()
    base = w * EPW
    pltpu.sync_copy(src_hbm.at[w], src_v)
    pltpu.sync_copy(dst_hbm.at[w], dst_v)

    def _step(j, _):
        cpa = pltpu.async_copy(a_hbm.at[src_v.at[j]], bufa, sema)
        cpb = pltpu.async_copy(b_hbm.at[dst_v.at[j]], bufb, semb)
        cpe = pltpu.async_copy(
            ea_hbm.at[pl.ds(base + j * CHUNK, CHUNK)], bufe, seme)
        cpa.wait()
        cpb.wait()
        cpe.wait()

        def _c(i, _):
            r = i // 8
            col = pl.ds((i % 8) * 16, 16)
            v = bufa[r, col] + bufb[r, col] + bufe[r, col]
            bufe[r, col] = jnp.maximum(v, 0.0)
            return 0
        lax.fori_loop(0, CHUNK * 8, _c, 0, unroll=8)
        pltpu.sync_copy(bufe, z_hbm.at[pl.ds(base + j * CHUNK, CHUNK)])
        return 0

    lax.fori_loop(0, NCHUNK, _step, 0)


_edge_call = pl.kernel(
    _edge_body,
    out_type=jax.ShapeDtypeStruct((E, D), jnp.float32),
    mesh=_MESH,
    compiler_params=_SC_PARAMS,
    scratch_types=[
        pltpu.VMEM((NCHUNK, CHUNK), jnp.int32),
        pltpu.VMEM((NCHUNK, CHUNK), jnp.int32),
        pltpu.VMEM((CHUNK, D), jnp.float32),
        pltpu.VMEM((CHUNK, D), jnp.float32),
        pltpu.VMEM((CHUNK, D), jnp.float32),
        pltpu.SemaphoreType.DMA,
        pltpu.SemaphoreType.DMA,
        pltpu.SemaphoreType.DMA,
    ],
)


_BN = 128
_BE = 512


def _tc1_body(hist_ref, x_ref, w1_ref, y1_ref, dis_ref):
    deg = jnp.sum(hist_ref[...], axis=0) + 1.0
    dis = lax.rsqrt(deg)
    xw = jnp.dot(x_ref[...], w1_ref[...], preferred_element_type=jnp.float32)
    y1_ref[...] = xw * dis
    dis_ref[...] = dis


def _tc2_body(sa_ref, sb_ref, y1_ref, dis_ref, b1_ref, w2_ref, y2_ref):
    dis = dis_ref[...]
    h1 = jnp.maximum(
        dis * (sa_ref[...] + sb_ref[...] + y1_ref[...]) + b1_ref[...], 0.0)
    y2_ref[...] = jnp.dot(h1, w2_ref[...],
                          preferred_element_type=jnp.float32) * dis


def _tc3_body(sa_ref, sb_ref, y2_ref, dis_ref, b2_ref, la_ref, lb_ref,
              a_ref, bm_ref):
    h2 = (dis_ref[...] * (sa_ref[...] + sb_ref[...] + y2_ref[...])
          + b2_ref[...])
    a_ref[...] = jnp.dot(h2, la_ref[...], preferred_element_type=jnp.float32)
    bm_ref[...] = jnp.dot(h2, lb_ref[...], preferred_element_type=jnp.float32)


def _tc3b_body(attr_ref, lc_ref, lb1_ref, ea_ref):
    ea_ref[...] = jnp.dot(attr_ref[...], lc_ref[...],
                          preferred_element_type=jnp.float32) + lb1_ref[...]


def _tc4_body(z_ref, w_ref, b_ref, o_ref):
    o_ref[...] = jnp.dot(z_ref[...], w_ref[...],
                         preferred_element_type=jnp.float32) + b_ref[...]


def _node_spec():
    return pl.BlockSpec((_BN, D), lambda i: (i, 0))


def _col_spec():
    return pl.BlockSpec((_BN, 1), lambda i: (i, 0))


def _full(shape):
    return pl.BlockSpec(shape, lambda i: tuple(0 for _ in shape))


def kernel(x, edge_index, edge_attr, W1, b1, W2, b2, lW1, lb1, lW2, lb2):
    f32 = jnp.float32
    src = edge_index[0]
    dst = edge_index[1]
    src2 = src.reshape(NW, NCHUNK, CHUNK)
    dst2 = dst.reshape(NW, NCHUNK, CHUNK)

    hist = _deg_call(dst)
    hist3 = hist.reshape(NW, N, 1)

    grid_n = pl.cdiv(N, _BN)
    y1, dis = pl.pallas_call(
        _tc1_body,
        grid=(grid_n,),
        in_specs=[pl.BlockSpec((NW, _BN, 1), lambda i: (0, i, 0)),
                  _node_spec(), _full((D, H))],
        out_specs=[_node_spec(), _col_spec()],
        out_shape=[jax.ShapeDtypeStruct((N, H), f32),
                   jax.ShapeDtypeStruct((N, 1), f32)],
    )(hist3, x, W1)

    s1 = _seg_call(y1, src, dst2)
    s1a, s1b = s1[:N], s1[N16:N16 + N]

    y2 = pl.pallas_call(
        _tc2_body,
        grid=(grid_n,),
        in_specs=[_node_spec(), _node_spec(), _node_spec(), _col_spec(),
                  _full((1, H)), _full((H, H))],
        out_specs=_node_spec(),
        out_shape=jax.ShapeDtypeStruct((N, H), f32),
    )(s1a, s1b, y1, dis, b1.reshape(1, H), W2)

    s2 = _seg_call(y2, src, dst2)
    s2a, s2b = s2[:N], s2[N16:N16 + N]

    a_n, b_n = pl.pallas_call(
        _tc3_body,
        grid=(grid_n,),
        in_specs=[_node_spec(), _node_spec(), _node_spec(), _col_spec(),
                  _full((1, H)), _full((H, H)), _full((H, H))],
        out_specs=[_node_spec(), _node_spec()],
        out_shape=[jax.ShapeDtypeStruct((N, H), f32),
                   jax.ShapeDtypeStruct((N, H), f32)],
    )(s2a, s2b, y2, dis, b2.reshape(1, H), lW1[:H], lW1[H:2 * H])

    ea = pl.pallas_call(
        _tc3b_body,
        grid=(E // _BE,),
        in_specs=[pl.BlockSpec((_BE, DE), lambda i: (i, 0)),
                  _full((DE, H)), _full((1, H))],
        out_specs=pl.BlockSpec((_BE, H), lambda i: (i, 0)),
        out_shape=jax.ShapeDtypeStruct((E, H), f32),
    )(edge_attr, lW1[2 * H:], lb1.reshape(1, H))

    z = _edge_call(a_n, b_n, ea, src2, dst2)

    out = pl.pallas_call(
        _tc4_body,
        grid=(E // _BE,),
        in_specs=[pl.BlockSpec((_BE, H), lambda i: (i, 0)),
                  _full((H, C)), _full((1, C))],
        out_specs=pl.BlockSpec((_BE, C), lambda i: (i, 0)),
        out_shape=jax.ShapeDtypeStruct((E, C), f32),
    )(z, lW2, lb2.reshape(1, C))
    return out


# speedup vs baseline: 9.8514x; 2.0928x over previous
import jax
import jax.numpy as jnp
from jax import lax
from jax.experimental import pallas as pl
from jax.experimental.pallas import tpu as pltpu
from jax.experimental.pallas import tpu_sc as plsc

N = 10000
E = 320000
D = 128
H = 128
DE = 16
C = 2

NC = 2
NS = 16
NW = NC * NS
EPW = E // NW
CHUNK = 80
NCHUNK = EPW // CHUNK
N16 = 10112
RPS = N16 // NS

_MESH = plsc.VectorSubcoreMesh(
    core_axis_name="c", subcore_axis_name="s", num_cores=NC, num_subcores=NS)
_SC_PARAMS = pltpu.CompilerParams(needs_layout_passes=False)


def _wid():
    return lax.axis_index("s") * NC + lax.axis_index("c")


def _deg_body(dst_hbm, hist_hbm, dst_v, hist_v):
    w = _wid()
    pltpu.sync_copy(dst_hbm.at[pl.ds(w * EPW, EPW)], dst_v)
    zeros = jnp.zeros((16,), jnp.float32)

    def _z(i, _):
        hist_v[pl.ds(i * 16, 16)] = zeros
        return 0
    lax.fori_loop(0, N // 16, _z, 0, unroll=8)

    ones = jnp.full((16,), 1.0, jnp.float32)

    def _acc(i, _):
        idx = dst_v[pl.ds(i * 16, 16)]
        plsc.addupdate_scatter(hist_v, [idx], ones)
        return 0
    lax.fori_loop(0, EPW // 16, _acc, 0, unroll=8)
    pltpu.sync_copy(hist_v, hist_hbm.at[pl.ds(w * N, N)])


_deg_call = pl.kernel(
    _deg_body,
    out_type=jax.ShapeDtypeStruct((NW * N,), jnp.float32),
    mesh=_MESH,
    compiler_params=_SC_PARAMS,
    scratch_types=[
        pltpu.VMEM((EPW,), jnp.int32),
        pltpu.VMEM((N,), jnp.float32),
    ],
)


def _seg_body(y_hbm, src_hbm, dst_hbm, out_hbm,
              src_v, dst_v, rows0, rows1, zbuf, acc, sem0, sem1):
    w = _wid()
    sid = lax.axis_index("s")
    cid = lax.axis_index("c")
    pltpu.sync_copy(src_hbm.at[pl.ds(w * EPW, EPW)], src_v)
    pltpu.sync_copy(dst_hbm.at[w], dst_v)

    zeros = jnp.zeros((16,), jnp.float32)

    def _z(i, _):
        zbuf[i // 8, pl.ds((i % 8) * 16, 16)] = zeros
        return 0
    lax.fori_loop(0, 64, _z, 0, unroll=8)

    def _zc(k, _):
        pltpu.sync_copy(zbuf, acc.at[pl.ds(sid * RPS + k * 8, 8)])
        return 0
    lax.fori_loop(0, RPS // 8, _zc, 0)
    plsc.subcore_barrier()

    def _sl(j):
        return src_v.at[pl.ds(j * CHUNK, CHUNK)]

    pltpu.async_copy(y_hbm.at[_sl(0)], rows0, sem0)

    def _step(j, _):
        even = j % 2 == 0

        @pl.when(j + 1 < NCHUNK)
        def _():
            @pl.when(even)
            def _():
                pltpu.async_copy(y_hbm.at[_sl(j + 1)], rows1, sem1)

            @pl.when(jnp.logical_not(even))
            def _():
                pltpu.async_copy(y_hbm.at[_sl(j + 1)], rows0, sem0)

        @pl.when(even)
        def _():
            pltpu.make_async_copy(y_hbm.at[_sl(j)], rows0, sem0).wait()
            pltpu.sync_copy(rows0, acc.at[dst_v.at[j]], add=True)

        @pl.when(jnp.logical_not(even))
        def _():
            pltpu.make_async_copy(y_hbm.at[_sl(j)], rows1, sem1).wait()
            pltpu.sync_copy(rows1, acc.at[dst_v.at[j]], add=True)
        return 0

    lax.fori_loop(0, NCHUNK, _step, 0)
    plsc.subcore_barrier()
    pltpu.sync_copy(acc.at[pl.ds(sid * RPS, RPS)],
                    out_hbm.at[pl.ds(cid * N16 + sid * RPS, RPS)])


_seg_call = pl.kernel(
    _seg_body,
    out_type=jax.ShapeDtypeStruct((NC * N16, D), jnp.float32),
    mesh=_MESH,
    compiler_params=_SC_PARAMS,
    scratch_types=[
        pltpu.VMEM((EPW,), jnp.int32),
        pltpu.VMEM((NCHUNK, CHUNK), jnp.int32),
        pltpu.VMEM((CHUNK, D), jnp.float32),
        pltpu.VMEM((CHUNK, D), jnp.float32),
        pltpu.VMEM((8, D), jnp.float32),
        pltpu.VMEM_SHARED((N16, D), jnp.float32),
        pltpu.SemaphoreType.DMA,
        pltpu.SemaphoreType.DMA,
    ],
)


def _edge_body(a_hbm, b_hbm, src_hbm, dst_hbm, z_hbm,
               src_v, dst_v, a0, a1, b0, b1,
               sa0, sa1, sb0, sb1):
    w = _wid()
    base = w * EPW
    pltpu.sync_copy(src_hbm.at[pl.ds(base, EPW)], src_v)
    pltpu.sync_copy(dst_hbm.at[pl.ds(base, EPW)], dst_v)

    def _sl(v, j):
        return v.at[pl.ds(j * CHUNK, CHUNK)]

    pltpu.async_copy(a_hbm.at[_sl(src_v, 0)], a0, sa0)
    pltpu.async_copy(b_hbm.at[_sl(dst_v, 0)], b0, sb0)

    def _sum_store(j, abuf, bbuf, asem, bsem):
        pltpu.make_async_copy(a_hbm.at[_sl(src_v, j)], abuf, asem).wait()
        pltpu.make_async_copy(b_hbm.at[_sl(dst_v, j)], bbuf, bsem).wait()

        def _c(r, _):
            for c in range(8):
                col = pl.ds(c * 16, 16)
                abuf[r, col] = abuf[r, col] + bbuf[r, col]
            return 0
        lax.fori_loop(0, CHUNK, _c, 0, unroll=4)
        pltpu.sync_copy(abuf, z_hbm.at[pl.ds(base + j * CHUNK, CHUNK)])

    def _step(j, _):
        even = j % 2 == 0

        @pl.when(j + 1 < NCHUNK)
        def _():
            @pl.when(even)
            def _():
                pltpu.async_copy(a_hbm.at[_sl(src_v, j + 1)], a1, sa1)
                pltpu.async_copy(b_hbm.at[_sl(dst_v, j + 1)], b1, sb1)

            @pl.when(jnp.logical_not(even))
            def _():
                pltpu.async_copy(a_hbm.at[_sl(src_v, j + 1)], a0, sa0)
                pltpu.async_copy(b_hbm.at[_sl(dst_v, j + 1)], b0, sb0)

        @pl.when(even)
        def _():
            _sum_store(j, a0, b0, sa0, sb0)

        @pl.when(jnp.logical_not(even))
        def _():
            _sum_store(j, a1, b1, sa1, sb1)
        return 0

    lax.fori_loop(0, NCHUNK, _step, 0)


_edge_call = pl.kernel(
    _edge_body,
    out_type=jax.ShapeDtypeStruct((E, D), jnp.float32),
    mesh=_MESH,
    compiler_params=_SC_PARAMS,
    scratch_types=[
        pltpu.VMEM((EPW,), jnp.int32),
        pltpu.VMEM((EPW,), jnp.int32),
        pltpu.VMEM((CHUNK, D), jnp.float32),
        pltpu.VMEM((CHUNK, D), jnp.float32),
        pltpu.VMEM((CHUNK, D), jnp.float32),
        pltpu.VMEM((CHUNK, D), jnp.float32),
        pltpu.SemaphoreType.DMA,
        pltpu.SemaphoreType.DMA,
        pltpu.SemaphoreType.DMA,
        pltpu.SemaphoreType.DMA,
    ],
)


_BN = 128
_BE = 2560


def _diag(v_row):
    ri = lax.broadcasted_iota(jnp.int32, (_BN, _BN), 0)
    ci = lax.broadcasted_iota(jnp.int32, (_BN, _BN), 1)
    eye = jnp.where(ri == ci, 1.0, 0.0).astype(jnp.float32)
    return eye * v_row


def _tc1_body(hist_ref, x_ref, w1_ref, y1_ref, disf_ref):
    deg = jnp.sum(hist_ref[...], axis=0, keepdims=True) + 1.0
    disd = _diag(lax.rsqrt(deg))
    xw = jnp.dot(x_ref[...], w1_ref[...], preferred_element_type=jnp.float32)
    y1_ref[...] = jnp.dot(disd, xw, preferred_element_type=jnp.float32)
    disf_ref[...] = jnp.dot(disd, jnp.ones((_BN, D), jnp.float32),
                            preferred_element_type=jnp.float32)


def _tc2_body(sa_ref, sb_ref, y1_ref, dis_ref, b1_ref, w2_ref, y2_ref):
    dis = dis_ref[...]
    h1 = jnp.maximum(
        dis * (sa_ref[...] + sb_ref[...] + y1_ref[...]) + b1_ref[...], 0.0)
    y2_ref[...] = jnp.dot(h1, w2_ref[...],
                          preferred_element_type=jnp.float32) * dis


def _tc3_body(sa_ref, sb_ref, y2_ref, dis_ref, b2_ref, la_ref, lb_ref,
              a_ref, bm_ref):
    h2 = (dis_ref[...] * (sa_ref[...] + sb_ref[...] + y2_ref[...])
          + b2_ref[...])
    a_ref[...] = jnp.dot(h2, la_ref[...], preferred_element_type=jnp.float32)
    bm_ref[...] = jnp.dot(h2, lb_ref[...], preferred_element_type=jnp.float32)


def _tc4_body(g_ref, attr_ref, lc_ref, lb1_ref, w_ref, b_ref, o_ref):
    e = jnp.maximum(
        g_ref[...] + jnp.dot(attr_ref[...], lc_ref[...],
                             preferred_element_type=jnp.float32)
        + lb1_ref[...], 0.0)
    o_ref[...] = jnp.dot(e, w_ref[...],
                         preferred_element_type=jnp.float32) + b_ref[...]


def _node_spec():
    return pl.BlockSpec((_BN, D), lambda i: (i, 0))


def _full(shape):
    return pl.BlockSpec(shape, lambda i: tuple(0 for _ in shape))


def kernel(x, edge_index, edge_attr, W1, b1, W2, b2, lW1, lb1, lW2, lb2):
    f32 = jnp.float32
    src = edge_index[0]
    dst = edge_index[1]
    dst2 = dst.reshape(NW, NCHUNK, CHUNK)

    hist = _deg_call(dst)
    hist2 = hist.reshape(NW, N)

    grid_n = pl.cdiv(N, _BN)
    y1, disf = pl.pallas_call(
        _tc1_body,
        grid=(grid_n,),
        in_specs=[pl.BlockSpec((NW, _BN), lambda i: (0, i)),
                  _node_spec(), _full((D, H))],
        out_specs=[_node_spec(), _node_spec()],
        out_shape=[jax.ShapeDtypeStruct((N, H), f32),
                   jax.ShapeDtypeStruct((N, H), f32)],
    )(hist2, x, W1)

    s1 = _seg_call(y1, src, dst2)
    _sa = pl.BlockSpec((_BN, D), lambda i: (i, 0))
    _sb = pl.BlockSpec((_BN, D), lambda i: (i + N16 // _BN, 0))

    y2 = pl.pallas_call(
        _tc2_body,
        grid=(grid_n,),
        in_specs=[_sa, _sb, _node_spec(), _node_spec(),
                  _full((1, H)), _full((H, H))],
        out_specs=_node_spec(),
        out_shape=jax.ShapeDtypeStruct((N, H), f32),
    )(s1, s1, y1, disf, b1.reshape(1, H), W2)

    s2 = _seg_call(y2, src, dst2)

    a_n, b_n = pl.pallas_call(
        _tc3_body,
        grid=(grid_n,),
        in_specs=[_sa, _sb, _node_spec(), _node_spec(),
                  _full((1, H)), _full((H, H)), _full((H, H))],
        out_specs=[_node_spec(), _node_spec()],
        out_shape=[jax.ShapeDtypeStruct((N, H), f32),
                   jax.ShapeDtypeStruct((N, H), f32)],
    )(s2, s2, y2, disf, b2.reshape(1, H), lW1[:H], lW1[H:2 * H])

    z = _edge_call(a_n, b_n, src, dst)

    out = pl.pallas_call(
        _tc4_body,
        grid=(E // _BE,),
        in_specs=[pl.BlockSpec((_BE, H), lambda i: (i, 0)),
                  pl.BlockSpec((_BE, DE), lambda i: (i, 0)),
                  _full((DE, H)), _full((1, H)), _full((H, C)),
                  _full((1, C))],
        out_specs=pl.BlockSpec((_BE, C), lambda i: (i, 0)),
        out_shape=jax.ShapeDtypeStruct((E, C), f32),
    )(z, edge_attr, lW1[2 * H:], lb1.reshape(1, H), lW2, lb2.reshape(1, C))
    return out


# speedup vs baseline: 10.7459x; 1.0908x over previous
import jax
import jax.numpy as jnp
from jax import lax
from jax.experimental import pallas as pl
from jax.experimental.pallas import tpu as pltpu
from jax.experimental.pallas import tpu_sc as plsc

N = 10000
E = 320000
D = 128
H = 128
DE = 16
C = 2

NC = 2
NS = 16
NW = NC * NS
EPW = E // NW
CHUNK = 80
NCHUNK = EPW // CHUNK
N16 = 10240
RPS = N16 // NS

_MESH = plsc.VectorSubcoreMesh(
    core_axis_name="c", subcore_axis_name="s", num_cores=NC, num_subcores=NS)
_SC_PARAMS = pltpu.CompilerParams(needs_layout_passes=False)


def _wid():
    return lax.axis_index("s") * NC + lax.axis_index("c")


def _deg_body(dst_hbm, hist_hbm, dst_v, hist_v):
    w = _wid()
    pltpu.sync_copy(dst_hbm.at[pl.ds(w * EPW, EPW)], dst_v)
    zeros = jnp.zeros((16,), jnp.float32)

    def _z(i, _):
        hist_v[pl.ds(i * 16, 16)] = zeros
        return 0
    lax.fori_loop(0, N // 16, _z, 0, unroll=8)

    ones = jnp.full((16,), 1.0, jnp.float32)

    def _acc(i, _):
        idx = dst_v[pl.ds(i * 16, 16)]
        plsc.addupdate_scatter(hist_v, [idx], ones)
        return 0
    lax.fori_loop(0, EPW // 16, _acc, 0, unroll=8)
    pltpu.sync_copy(hist_v, hist_hbm.at[pl.ds(w * N, N)])


_deg_call = pl.kernel(
    _deg_body,
    out_type=jax.ShapeDtypeStruct((NW * N,), jnp.float32),
    mesh=_MESH,
    compiler_params=_SC_PARAMS,
    scratch_types=[
        pltpu.VMEM((EPW,), jnp.int32),
        pltpu.VMEM((N,), jnp.float32),
    ],
)


def _seg_body(y_hbm, src_hbm, dst_hbm, out_hbm,
              src_v, dst_v, rows0, rows1, zbuf, acc, sem0, sem1):
    w = _wid()
    sid = lax.axis_index("s")
    cid = lax.axis_index("c")
    pltpu.sync_copy(src_hbm.at[pl.ds(w * EPW, EPW)], src_v)
    pltpu.sync_copy(dst_hbm.at[w], dst_v)

    zeros = jnp.zeros((16,), jnp.float32)

    def _z(i, _):
        zbuf[i // 8, pl.ds((i % 8) * 16, 16)] = zeros
        return 0
    lax.fori_loop(0, 64, _z, 0, unroll=8)

    def _zc(k, _):
        pltpu.sync_copy(zbuf, acc.at[pl.ds(sid * RPS + k * 8, 8)])
        return 0
    lax.fori_loop(0, RPS // 8, _zc, 0)
    plsc.subcore_barrier()

    def _sl(j):
        return src_v.at[pl.ds(j * CHUNK, CHUNK)]

    pltpu.async_copy(y_hbm.at[_sl(0)], rows0, sem0)

    def _step(j, _):
        even = j % 2 == 0

        @pl.when(j + 1 < NCHUNK)
        def _():
            @pl.when(even)
            def _():
                pltpu.async_copy(y_hbm.at[_sl(j + 1)], rows1, sem1)

            @pl.when(jnp.logical_not(even))
            def _():
                pltpu.async_copy(y_hbm.at[_sl(j + 1)], rows0, sem0)

        @pl.when(even)
        def _():
            pltpu.make_async_copy(y_hbm.at[_sl(j)], rows0, sem0).wait()
            pltpu.sync_copy(rows0, acc.at[dst_v.at[j]], add=True)

        @pl.when(jnp.logical_not(even))
        def _():
            pltpu.make_async_copy(y_hbm.at[_sl(j)], rows1, sem1).wait()
            pltpu.sync_copy(rows1, acc.at[dst_v.at[j]], add=True)
        return 0

    lax.fori_loop(0, NCHUNK, _step, 0)
    plsc.subcore_barrier()
    pltpu.sync_copy(acc.at[pl.ds(sid * RPS, RPS)],
                    out_hbm.at[pl.ds(cid * N16 + sid * RPS, RPS)])


_seg_call = pl.kernel(
    _seg_body,
    out_type=jax.ShapeDtypeStruct((NC * N16, D), jnp.float32),
    mesh=_MESH,
    compiler_params=_SC_PARAMS,
    scratch_types=[
        pltpu.VMEM((EPW,), jnp.int32),
        pltpu.VMEM((NCHUNK, CHUNK), jnp.int32),
        pltpu.VMEM((CHUNK, D), jnp.float32),
        pltpu.VMEM((CHUNK, D), jnp.float32),
        pltpu.VMEM((8, D), jnp.float32),
        pltpu.VMEM_SHARED((N16, D), jnp.float32),
        pltpu.SemaphoreType.DMA,
        pltpu.SemaphoreType.DMA,
    ],
)


def _edge_body(a_hbm, b_hbm, src_hbm, dst_hbm, z_hbm,
               src_v, dst_v, a0, a1, b0, b1,
               sa0, sa1, sb0, sb1, so0, so1):
    w = _wid()
    base = w * EPW
    pltpu.sync_copy(src_hbm.at[pl.ds(base, EPW)], src_v)
    pltpu.sync_copy(dst_hbm.at[pl.ds(base, EPW)], dst_v)

    def _sl(v, j):
        return v.at[pl.ds(j * CHUNK, CHUNK)]

    def _zsl(j):
        return z_hbm.at[pl.ds(base + j * CHUNK, CHUNK)]

    pltpu.async_copy(a_hbm.at[_sl(src_v, 0)], a0, sa0)
    pltpu.async_copy(b_hbm.at[_sl(dst_v, 0)], b0, sb0)

    def _proc(j, abuf, bbuf, asem, bsem, osem):
        pltpu.make_async_copy(a_hbm.at[_sl(src_v, j)], abuf, asem).wait()
        pltpu.make_async_copy(b_hbm.at[_sl(dst_v, j)], bbuf, bsem).wait()

        def _c(r, _):
            for c in range(8):
                col = pl.ds(c * 16, 16)
                abuf[r, col] = abuf[r, col] + bbuf[r, col]
            return 0
        lax.fori_loop(0, CHUNK, _c, 0, unroll=8)
        pltpu.async_copy(abuf, _zsl(j), osem)

    def _wait_store(abuf, osem):
        pltpu.make_async_copy(abuf, z_hbm.at[pl.ds(base, CHUNK)], osem).wait()

    def _step(j, _):
        even = j % 2 == 0

        @pl.when(jnp.logical_and(even, j + 1 < NCHUNK))
        def _():
            @pl.when(j >= 2)
            def _():
                _wait_store(a1, so1)
            pltpu.async_copy(a_hbm.at[_sl(src_v, j + 1)], a1, sa1)
            pltpu.async_copy(b_hbm.at[_sl(dst_v, j + 1)], b1, sb1)

        @pl.when(jnp.logical_and(jnp.logical_not(even), j + 1 < NCHUNK))
        def _():
            _wait_store(a0, so0)
            pltpu.async_copy(a_hbm.at[_sl(src_v, j + 1)], a0, sa0)
            pltpu.async_copy(b_hbm.at[_sl(dst_v, j + 1)], b0, sb0)

        @pl.when(even)
        def _():
            _proc(j, a0, b0, sa0, sb0, so0)

        @pl.when(jnp.logical_not(even))
        def _():
            _proc(j, a1, b1, sa1, sb1, so1)
        return 0

    lax.fori_loop(0, NCHUNK, _step, 0)
    _wait_store(a0, so0)
    _wait_store(a1, so1)


_edge_call = pl.kernel(
    _edge_body,
    out_type=jax.ShapeDtypeStruct((E, D), jnp.float32),
    mesh=_MESH,
    compiler_params=_SC_PARAMS,
    scratch_types=[
        pltpu.VMEM((EPW,), jnp.int32),
        pltpu.VMEM((EPW,), jnp.int32),
        pltpu.VMEM((CHUNK, D), jnp.float32),
        pltpu.VMEM((CHUNK, D), jnp.float32),
        pltpu.VMEM((CHUNK, D), jnp.float32),
        pltpu.VMEM((CHUNK, D), jnp.float32),
        pltpu.SemaphoreType.DMA,
        pltpu.SemaphoreType.DMA,
        pltpu.SemaphoreType.DMA,
        pltpu.SemaphoreType.DMA,
        pltpu.SemaphoreType.DMA,
        pltpu.SemaphoreType.DMA,
    ],
)


_BN = 512
_BE = 2560


def _diag(v_row):
    ri = lax.broadcasted_iota(jnp.int32, (_BN, _BN), 0)
    ci = lax.broadcasted_iota(jnp.int32, (_BN, _BN), 1)
    eye = jnp.where(ri == ci, 1.0, 0.0).astype(jnp.float32)
    return eye * v_row


def _tc1_body(hist_ref, x_ref, w1_ref, y1_ref, disf_ref):
    deg = jnp.sum(hist_ref[...], axis=0, keepdims=True) + 1.0
    disd = _diag(lax.rsqrt(deg))
    xw = jnp.dot(x_ref[...], w1_ref[...], preferred_element_type=jnp.float32)
    y1_ref[...] = jnp.dot(disd, xw, preferred_element_type=jnp.float32)
    disf_ref[...] = jnp.dot(disd, jnp.ones((_BN, D), jnp.float32),
                            preferred_element_type=jnp.float32)


def _tc2_body(sa_ref, sb_ref, y1_ref, dis_ref, b1_ref, w2_ref, y2_ref):
    dis = dis_ref[...]
    h1 = jnp.maximum(
        dis * (sa_ref[...] + sb_ref[...] + y1_ref[...]) + b1_ref[...], 0.0)
    y2_ref[...] = jnp.dot(h1, w2_ref[...],
                          preferred_element_type=jnp.float32) * dis


def _tc3_body(sa_ref, sb_ref, y2_ref, dis_ref, b2_ref, la_ref, lb_ref,
              a_ref, bm_ref):
    h2 = (dis_ref[...] * (sa_ref[...] + sb_ref[...] + y2_ref[...])
          + b2_ref[...])
    a_ref[...] = jnp.dot(h2, la_ref[...], preferred_element_type=jnp.float32)
    bm_ref[...] = jnp.dot(h2, lb_ref[...], preferred_element_type=jnp.float32)


def _tc4_body(g_ref, attr_ref, lc_ref, lb1_ref, w_ref, b_ref, o_ref):
    e = jnp.maximum(
        g_ref[...] + jnp.dot(attr_ref[...], lc_ref[...],
                             preferred_element_type=jnp.float32)
        + lb1_ref[...], 0.0)
    o_ref[...] = jnp.dot(e, w_ref[...],
                         preferred_element_type=jnp.float32) + b_ref[...]


def _node_spec():
    return pl.BlockSpec((_BN, D), lambda i: (i, 0))


def _full(shape):
    return pl.BlockSpec(shape, lambda i: tuple(0 for _ in shape))


def kernel(x, edge_index, edge_attr, W1, b1, W2, b2, lW1, lb1, lW2, lb2):
    f32 = jnp.float32
    src = edge_index[0]
    dst = edge_index[1]
    dst2 = dst.reshape(NW, NCHUNK, CHUNK)

    hist = _deg_call(dst)
    hist2 = hist.reshape(NW, N)

    grid_n = pl.cdiv(N, _BN)
    y1, disf = pl.pallas_call(
        _tc1_body,
        grid=(grid_n,),
        in_specs=[pl.BlockSpec((NW, _BN), lambda i: (0, i)),
                  _node_spec(), _full((D, H))],
        out_specs=[_node_spec(), _node_spec()],
        out_shape=[jax.ShapeDtypeStruct((N, H), f32),
                   jax.ShapeDtypeStruct((N, H), f32)],
    )(hist2, x, W1)

    s1 = _seg_call(y1, src, dst2)
    _sa = pl.BlockSpec((_BN, D), lambda i: (i, 0))
    _sb = pl.BlockSpec((_BN, D), lambda i: (i + N16 // _BN, 0))

    y2 = pl.pallas_call(
        _tc2_body,
        grid=(grid_n,),
        in_specs=[_sa, _sb, _node_spec(), _node_spec(),
                  _full((1, H)), _full((H, H))],
        out_specs=_node_spec(),
        out_shape=jax.ShapeDtypeStruct((N, H), f32),
    )(s1, s1, y1, disf, b1.reshape(1, H), W2)

    s2 = _seg_call(y2, src, dst2)

    a_n, b_n = pl.pallas_call(
        _tc3_body,
        grid=(grid_n,),
        in_specs=[_sa, _sb, _node_spec(), _node_spec(),
                  _full((1, H)), _full((H, H)), _full((H, H))],
        out_specs=[_node_spec(), _node_spec()],
        out_shape=[jax.ShapeDtypeStruct((N, H), f32),
                   jax.ShapeDtypeStruct((N, H), f32)],
    )(s2, s2, y2, disf, b2.reshape(1, H), lW1[:H], lW1[H:2 * H])

    z = _edge_call(a_n, b_n, src, dst)

    out = pl.pallas_call(
        _tc4_body,
        grid=(E // _BE,),
        in_specs=[pl.BlockSpec((_BE, H), lambda i: (i, 0)),
                  pl.BlockSpec((_BE, DE), lambda i: (i, 0)),
                  _full((DE, H)), _full((1, H)), _full((H, C)),
                  _full((1, C))],
        out_specs=pl.BlockSpec((_BE, C), lambda i: (i, 0)),
        out_shape=jax.ShapeDtypeStruct((E, C), f32),
    )(z, edge_attr, lW1[2 * H:], lb1.reshape(1, H), lW2, lb2.reshape(1, C))
    return out


# speedup vs baseline: 13.1461x; 1.2234x over previous
import jax
import jax.numpy as jnp
from jax import lax
from jax.experimental import pallas as pl
from jax.experimental.pallas import tpu as pltpu
from jax.experimental.pallas import tpu_sc as plsc

N = 10000
E = 320000
D = 128
H = 128
DE = 16
C = 2

NC = 2
NS = 16
NW = NC * NS
EPW = E // NW
CHUNK = 80
NCHUNK = EPW // CHUNK
N16 = 10240
RPS = N16 // NS

_MESH = plsc.VectorSubcoreMesh(
    core_axis_name="c", subcore_axis_name="s", num_cores=NC, num_subcores=NS)
_SC_PARAMS = pltpu.CompilerParams(needs_layout_passes=False)


def _wid():
    return lax.axis_index("s") * NC + lax.axis_index("c")


def _deg_body(dst_hbm, hist_hbm, dst_v, hist_v):
    w = _wid()
    pltpu.sync_copy(dst_hbm.at[pl.ds(w * EPW, EPW)], dst_v)
    zeros = jnp.zeros((16,), jnp.float32)

    def _z(i, _):
        hist_v[pl.ds(i * 16, 16)] = zeros
        return 0
    lax.fori_loop(0, N // 16, _z, 0, unroll=8)

    ones = jnp.full((16,), 1.0, jnp.float32)

    def _acc(i, _):
        idx = dst_v[pl.ds(i * 16, 16)]
        plsc.addupdate_scatter(hist_v, [idx], ones)
        return 0
    lax.fori_loop(0, EPW // 16, _acc, 0, unroll=8)
    pltpu.sync_copy(hist_v, hist_hbm.at[pl.ds(w * N, N)])


_deg_call = pl.kernel(
    _deg_body,
    out_type=jax.ShapeDtypeStruct((NW * N,), jnp.float32),
    mesh=_MESH,
    compiler_params=_SC_PARAMS,
    scratch_types=[
        pltpu.VMEM((EPW,), jnp.int32),
        pltpu.VMEM((N,), jnp.float32),
    ],
)


def _seg_body(y_hbm, src_hbm, dst_hbm, out_hbm,
              src_v, dst_v, rows0, rows1, zbuf, acc, sem0, sem1):
    w = _wid()
    sid = lax.axis_index("s")
    cid = lax.axis_index("c")
    pltpu.sync_copy(src_hbm.at[pl.ds(w * EPW, EPW)], src_v)
    pltpu.sync_copy(dst_hbm.at[w], dst_v)

    zeros = jnp.zeros((16,), jnp.float32)

    def _z(i, _):
        zbuf[i // 8, pl.ds((i % 8) * 16, 16)] = zeros
        return 0
    lax.fori_loop(0, 64, _z, 0, unroll=8)

    def _zc(k, _):
        pltpu.sync_copy(zbuf, acc.at[pl.ds(sid * RPS + k * 8, 8)])
        return 0
    lax.fori_loop(0, RPS // 8, _zc, 0)
    plsc.subcore_barrier()

    def _sl(j):
        return src_v.at[pl.ds(j * CHUNK, CHUNK)]

    pltpu.async_copy(y_hbm.at[_sl(0)], rows0, sem0)

    def _step(j, _):
        even = j % 2 == 0

        @pl.when(j + 1 < NCHUNK)
        def _():
            @pl.when(even)
            def _():
                pltpu.async_copy(y_hbm.at[_sl(j + 1)], rows1, sem1)

            @pl.when(jnp.logical_not(even))
            def _():
                pltpu.async_copy(y_hbm.at[_sl(j + 1)], rows0, sem0)

        @pl.when(even)
        def _():
            pltpu.make_async_copy(y_hbm.at[_sl(j)], rows0, sem0).wait()
            pltpu.sync_copy(rows0, acc.at[dst_v.at[j]], add=True)

        @pl.when(jnp.logical_not(even))
        def _():
            pltpu.make_async_copy(y_hbm.at[_sl(j)], rows1, sem1).wait()
            pltpu.sync_copy(rows1, acc.at[dst_v.at[j]], add=True)
        return 0

    lax.fori_loop(0, NCHUNK, _step, 0)
    plsc.subcore_barrier()
    pltpu.sync_copy(acc.at[pl.ds(sid * RPS, RPS)],
                    out_hbm.at[pl.ds(cid * N16 + sid * RPS, RPS)])


_seg_call = pl.kernel(
    _seg_body,
    out_type=jax.ShapeDtypeStruct((NC * N16, D), jnp.float32),
    mesh=_MESH,
    compiler_params=_SC_PARAMS,
    scratch_types=[
        pltpu.VMEM((EPW,), jnp.int32),
        pltpu.VMEM((NCHUNK, CHUNK), jnp.int32),
        pltpu.VMEM((CHUNK, D), jnp.float32),
        pltpu.VMEM((CHUNK, D), jnp.float32),
        pltpu.VMEM((8, D), jnp.float32),
        pltpu.VMEM_SHARED((N16, D), jnp.float32),
        pltpu.SemaphoreType.DMA,
        pltpu.SemaphoreType.DMA,
    ],
)


def _edge_body(a_hbm, b_hbm, src_hbm, dst_hbm, z_hbm,
               src_v, dst_v, a0, a1, b0, b1, o0, o1,
               sa0, sa1, sb0, sb1, so0, so1):
    w = _wid()
    base = w * EPW
    pltpu.sync_copy(src_hbm.at[pl.ds(base, EPW)], src_v)
    pltpu.sync_copy(dst_hbm.at[pl.ds(base, EPW)], dst_v)

    def _sl(v, j):
        return v.at[pl.ds(j * CHUNK, CHUNK)]

    def _zsl(j):
        return z_hbm.at[pl.ds(base + j * CHUNK, CHUNK)]

    pltpu.async_copy(a_hbm.at[_sl(src_v, 0)], a0, sa0)
    pltpu.async_copy(b_hbm.at[_sl(dst_v, 0)], b0, sb0)

    def _wait_store(obuf, osem):
        pltpu.make_async_copy(obuf, z_hbm.at[pl.ds(base, CHUNK)], osem).wait()

    def _proc(j, abuf, bbuf, obuf, asem, bsem, osem):
        @pl.when(j >= 2)
        def _():
            _wait_store(obuf, osem)
        pltpu.make_async_copy(a_hbm.at[_sl(src_v, j)], abuf, asem).wait()
        pltpu.make_async_copy(b_hbm.at[_sl(dst_v, j)], bbuf, bsem).wait()

        @plsc.parallel_loop(0, CHUNK, step=1, unroll=8)
        def _c(r):
            for c in range(8):
                col = pl.ds(c * 16, 16)
                obuf[r, col] = abuf[r, col] + bbuf[r, col]
        pltpu.async_copy(obuf, _zsl(j), osem)

    def _step(j, _):
        even = j % 2 == 0

        @pl.when(jnp.logical_and(even, j + 1 < NCHUNK))
        def _():
            pltpu.async_copy(a_hbm.at[_sl(src_v, j + 1)], a1, sa1)
            pltpu.async_copy(b_hbm.at[_sl(dst_v, j + 1)], b1, sb1)

        @pl.when(jnp.logical_and(jnp.logical_not(even), j + 1 < NCHUNK))
        def _():
            pltpu.async_copy(a_hbm.at[_sl(src_v, j + 1)], a0, sa0)
            pltpu.async_copy(b_hbm.at[_sl(dst_v, j + 1)], b0, sb0)

        @pl.when(even)
        def _():
            _proc(j, a0, b0, o0, sa0, sb0, so0)

        @pl.when(jnp.logical_not(even))
        def _():
            _proc(j, a1, b1, o1, sa1, sb1, so1)
        return 0

    lax.fori_loop(0, NCHUNK, _step, 0)
    _wait_store(o0, so0)
    _wait_store(o1, so1)


_edge_call = pl.kernel(
    _edge_body,
    out_type=jax.ShapeDtypeStruct((E, D), jnp.float32),
    mesh=_MESH,
    compiler_params=_SC_PARAMS,
    scratch_types=[
        pltpu.VMEM((EPW,), jnp.int32),
        pltpu.VMEM((EPW,), jnp.int32),
        pltpu.VMEM((CHUNK, D), jnp.float32),
        pltpu.VMEM((CHUNK, D), jnp.float32),
        pltpu.VMEM((CHUNK, D), jnp.float32),
        pltpu.VMEM((CHUNK, D), jnp.float32),
        pltpu.VMEM((CHUNK, D), jnp.float32),
        pltpu.VMEM((CHUNK, D), jnp.float32),
        pltpu.SemaphoreType.DMA,
        pltpu.SemaphoreType.DMA,
        pltpu.SemaphoreType.DMA,
        pltpu.SemaphoreType.DMA,
        pltpu.SemaphoreType.DMA,
        pltpu.SemaphoreType.DMA,
    ],
)


_BN = 512
_BE = 2560


def _diag(v_row):
    ri = lax.broadcasted_iota(jnp.int32, (_BN, _BN), 0)
    ci = lax.broadcasted_iota(jnp.int32, (_BN, _BN), 1)
    eye = jnp.where(ri == ci, 1.0, 0.0).astype(jnp.float32)
    return eye * v_row


def _tc1_body(hist_ref, x_ref, w1_ref, y1_ref, disf_ref):
    deg = jnp.sum(hist_ref[...], axis=0, keepdims=True) + 1.0
    disd = _diag(lax.rsqrt(deg))
    xw = jnp.dot(x_ref[...], w1_ref[...], preferred_element_type=jnp.float32)
    y1_ref[...] = jnp.dot(disd, xw, preferred_element_type=jnp.float32)
    disf_ref[...] = jnp.dot(disd, jnp.ones((_BN, D), jnp.float32),
                            preferred_element_type=jnp.float32)


def _tc2_body(sa_ref, sb_ref, y1_ref, dis_ref, b1_ref, w2_ref, y2_ref):
    dis = dis_ref[...]
    h1 = jnp.maximum(
        dis * (sa_ref[...] + sb_ref[...] + y1_ref[...]) + b1_ref[...], 0.0)
    y2_ref[...] = jnp.dot(h1, w2_ref[...],
                          preferred_element_type=jnp.float32) * dis


def _tc3_body(sa_ref, sb_ref, y2_ref, dis_ref, b2_ref, la_ref, lb_ref,
              a_ref, bm_ref):
    h2 = (dis_ref[...] * (sa_ref[...] + sb_ref[...] + y2_ref[...])
          + b2_ref[...])
    a_ref[...] = jnp.dot(h2, la_ref[...], preferred_element_type=jnp.float32)
    bm_ref[...] = jnp.dot(h2, lb_ref[...], preferred_element_type=jnp.float32)


def _tc4_body(g_ref, attr_ref, lc_ref, lb1_ref, w_ref, b_ref, o_ref):
    e = jnp.maximum(
        g_ref[...] + jnp.dot(attr_ref[...], lc_ref[...],
                             preferred_element_type=jnp.float32)
        + lb1_ref[...], 0.0)
    o_ref[...] = jnp.dot(e, w_ref[...],
                         preferred_element_type=jnp.float32) + b_ref[...]


def _node_spec():
    return pl.BlockSpec((_BN, D), lambda i: (i, 0))


def _full(shape):
    return pl.BlockSpec(shape, lambda i: tuple(0 for _ in shape))


def kernel(x, edge_index, edge_attr, W1, b1, W2, b2, lW1, lb1, lW2, lb2):
    f32 = jnp.float32
    src = edge_index[0]
    dst = edge_index[1]
    dst2 = dst.reshape(NW, NCHUNK, CHUNK)

    hist = _deg_call(dst)
    hist2 = hist.reshape(NW, N)

    grid_n = pl.cdiv(N, _BN)
    y1, disf = pl.pallas_call(
        _tc1_body,
        grid=(grid_n,),
        in_specs=[pl.BlockSpec((NW, _BN), lambda i: (0, i)),
                  _node_spec(), _full((D, H))],
        out_specs=[_node_spec(), _node_spec()],
        out_shape=[jax.ShapeDtypeStruct((N, H), f32),
                   jax.ShapeDtypeStruct((N, H), f32)],
    )(hist2, x, W1)

    s1 = _seg_call(y1, src, dst2)
    _sa = pl.BlockSpec((_BN, D), lambda i: (i, 0))
    _sb = pl.BlockSpec((_BN, D), lambda i: (i + N16 // _BN, 0))

    y2 = pl.pallas_call(
        _tc2_body,
        grid=(grid_n,),
        in_specs=[_sa, _sb, _node_spec(), _node_spec(),
                  _full((1, H)), _full((H, H))],
        out_specs=_node_spec(),
        out_shape=jax.ShapeDtypeStruct((N, H), f32),
    )(s1, s1, y1, disf, b1.reshape(1, H), W2)

    s2 = _seg_call(y2, src, dst2)

    a_n, b_n = pl.pallas_call(
        _tc3_body,
        grid=(grid_n,),
        in_specs=[_sa, _sb, _node_spec(), _node_spec(),
                  _full((1, H)), _full((H, H)), _full((H, H))],
        out_specs=[_node_spec(), _node_spec()],
        out_shape=[jax.ShapeDtypeStruct((N, H), f32),
                   jax.ShapeDtypeStruct((N, H), f32)],
    )(s2, s2, y2, disf, b2.reshape(1, H), lW1[:H], lW1[H:2 * H])

    z = _edge_call(a_n, b_n, src, dst)

    out = pl.pallas_call(
        _tc4_body,
        grid=(E // _BE,),
        in_specs=[pl.BlockSpec((_BE, H), lambda i: (i, 0)),
                  pl.BlockSpec((_BE, DE), lambda i: (i, 0)),
                  _full((DE, H)), _full((1, H)), _full((H, C)),
                  _full((1, C))],
        out_specs=pl.BlockSpec((_BE, C), lambda i: (i, 0)),
        out_shape=jax.ShapeDtypeStruct((E, C), f32),
    )(z, edge_attr, lW1[2 * H:], lb1.reshape(1, H), lW2, lb2.reshape(1, C))
    return out


# speedup vs baseline: 16.9053x; 1.2860x over previous
import jax
import jax.numpy as jnp
from jax import lax
from jax.experimental import pallas as pl
from jax.experimental.pallas import tpu as pltpu
from jax.experimental.pallas import tpu_sc as plsc

N = 10000
E = 320000
D = 128
H = 128
DE = 16
C = 2

NC = 2
NS = 16
NW = NC * NS
EPW = E // NW
CHUNK = 80
NCHUNK = EPW // CHUNK
N16 = 10240
RPS = N16 // NS

_MESH = plsc.VectorSubcoreMesh(
    core_axis_name="c", subcore_axis_name="s", num_cores=NC, num_subcores=NS)
_SC_PARAMS = pltpu.CompilerParams(needs_layout_passes=False)


def _wid():
    return lax.axis_index("s") * NC + lax.axis_index("c")


def _deg_body(dst_hbm, hist_hbm, dst_v, hist_v):
    w = _wid()
    pltpu.sync_copy(dst_hbm.at[pl.ds(w * EPW, EPW)], dst_v)
    zeros = jnp.zeros((16,), jnp.float32)

    def _z(i, _):
        hist_v[pl.ds(i * 16, 16)] = zeros
        return 0
    lax.fori_loop(0, N // 16, _z, 0, unroll=8)

    ones = jnp.full((16,), 1.0, jnp.float32)

    def _acc(i, _):
        idx = dst_v[pl.ds(i * 16, 16)]
        plsc.addupdate_scatter(hist_v, [idx], ones)
        return 0
    lax.fori_loop(0, EPW // 16, _acc, 0, unroll=8)
    pltpu.sync_copy(hist_v, hist_hbm.at[pl.ds(w * N, N)])


_deg_call = pl.kernel(
    _deg_body,
    out_type=jax.ShapeDtypeStruct((NW * N,), jnp.float32),
    mesh=_MESH,
    compiler_params=_SC_PARAMS,
    scratch_types=[
        pltpu.VMEM((EPW,), jnp.int32),
        pltpu.VMEM((N,), jnp.float32),
    ],
)


def _seg_body(y_hbm, src_hbm, dst_hbm, out_hbm,
              src_v, dst_v, rows0, rows1, zbuf, acc, sem0, sem1):
    w = _wid()
    sid = lax.axis_index("s")
    cid = lax.axis_index("c")
    pltpu.sync_copy(src_hbm.at[pl.ds(w * EPW, EPW)], src_v)
    pltpu.sync_copy(dst_hbm.at[w], dst_v)

    zeros = jnp.zeros((16,), jnp.float32)

    def _z(i, _):
        zbuf[i // 8, pl.ds((i % 8) * 16, 16)] = zeros
        return 0
    lax.fori_loop(0, 64, _z, 0, unroll=8)

    def _zc(k, _):
        pltpu.sync_copy(zbuf, acc.at[pl.ds(sid * RPS + k * 8, 8)])
        return 0
    lax.fori_loop(0, RPS // 8, _zc, 0)
    plsc.subcore_barrier()

    def _sl(j):
        return src_v.at[pl.ds(j * CHUNK, CHUNK)]

    pltpu.async_copy(y_hbm.at[_sl(0)], rows0, sem0)

    def _step(j, _):
        even = j % 2 == 0

        @pl.when(j + 1 < NCHUNK)
        def _():
            @pl.when(even)
            def _():
                pltpu.async_copy(y_hbm.at[_sl(j + 1)], rows1, sem1)

            @pl.when(jnp.logical_not(even))
            def _():
                pltpu.async_copy(y_hbm.at[_sl(j + 1)], rows0, sem0)

        @pl.when(even)
        def _():
            pltpu.make_async_copy(y_hbm.at[_sl(j)], rows0, sem0).wait()
            pltpu.sync_copy(rows0, acc.at[dst_v.at[j]], add=True)

        @pl.when(jnp.logical_not(even))
        def _():
            pltpu.make_async_copy(y_hbm.at[_sl(j)], rows1, sem1).wait()
            pltpu.sync_copy(rows1, acc.at[dst_v.at[j]], add=True)
        return 0

    lax.fori_loop(0, NCHUNK, _step, 0)
    plsc.subcore_barrier()
    pltpu.sync_copy(acc.at[pl.ds(sid * RPS, RPS)],
                    out_hbm.at[pl.ds(cid * N16 + sid * RPS, RPS)])


_seg_call = pl.kernel(
    _seg_body,
    out_type=jax.ShapeDtypeStruct((NC * N16, D), jnp.float32),
    mesh=_MESH,
    compiler_params=_SC_PARAMS,
    scratch_types=[
        pltpu.VMEM((EPW,), jnp.int32),
        pltpu.VMEM((NCHUNK, CHUNK), jnp.int32),
        pltpu.VMEM((CHUNK, D), jnp.float32),
        pltpu.VMEM((CHUNK, D), jnp.float32),
        pltpu.VMEM((8, D), jnp.float32),
        pltpu.VMEM_SHARED((N16, D), jnp.float32),
        pltpu.SemaphoreType.DMA,
        pltpu.SemaphoreType.DMA,
    ],
)


def _edge_body(a_hbm, b_hbm, src_hbm, dst_hbm, z_hbm,
               src_v, dst_v, a0, a1, b0, b1, o0, o1,
               sa0, sa1, sb0, sb1, so0, so1):
    w = _wid()
    base = w * EPW
    pltpu.sync_copy(src_hbm.at[pl.ds(base, EPW)], src_v)
    pltpu.sync_copy(dst_hbm.at[pl.ds(base, EPW)], dst_v)

    def _sl(v, j):
        return v.at[pl.ds(j * CHUNK, CHUNK)]

    def _zsl(j):
        return z_hbm.at[pl.ds(base + j * CHUNK, CHUNK)]

    pltpu.async_copy(a_hbm.at[_sl(src_v, 0)], a0, sa0)
    pltpu.async_copy(b_hbm.at[_sl(dst_v, 0)], b0, sb0)

    def _wait_store(obuf, osem):
        pltpu.make_async_copy(obuf, z_hbm.at[pl.ds(base, CHUNK)], osem).wait()

    def _proc(j, abuf, bbuf, obuf, asem, bsem, osem):
        @pl.when(j >= 2)
        def _():
            _wait_store(obuf, osem)
        pltpu.make_async_copy(a_hbm.at[_sl(src_v, j)], abuf, asem).wait()
        pltpu.make_async_copy(b_hbm.at[_sl(dst_v, j)], bbuf, bsem).wait()

        @plsc.parallel_loop(0, CHUNK, step=1, unroll=8)
        def _c(r):
            for c in range(8):
                col = pl.ds(c * 16, 16)
                obuf[r, col] = abuf[r, col] + bbuf[r, col]
        pltpu.async_copy(obuf, _zsl(j), osem)

    def _step(j, _):
        even = j % 2 == 0

        @pl.when(jnp.logical_and(even, j + 1 < NCHUNK))
        def _():
            pltpu.async_copy(a_hbm.at[_sl(src_v, j + 1)], a1, sa1)
            pltpu.async_copy(b_hbm.at[_sl(dst_v, j + 1)], b1, sb1)

        @pl.when(jnp.logical_and(jnp.logical_not(even), j + 1 < NCHUNK))
        def _():
            pltpu.async_copy(a_hbm.at[_sl(src_v, j + 1)], a0, sa0)
            pltpu.async_copy(b_hbm.at[_sl(dst_v, j + 1)], b0, sb0)

        @pl.when(even)
        def _():
            _proc(j, a0, b0, o0, sa0, sb0, so0)

        @pl.when(jnp.logical_not(even))
        def _():
            _proc(j, a1, b1, o1, sa1, sb1, so1)
        return 0

    lax.fori_loop(0, NCHUNK, _step, 0)
    _wait_store(o0, so0)
    _wait_store(o1, so1)


_edge_call = pl.kernel(
    _edge_body,
    out_type=jax.ShapeDtypeStruct((E, D), jnp.float32),
    mesh=_MESH,
    compiler_params=_SC_PARAMS,
    scratch_types=[
        pltpu.VMEM((EPW,), jnp.int32),
        pltpu.VMEM((EPW,), jnp.int32),
        pltpu.VMEM((CHUNK, D), jnp.float32),
        pltpu.VMEM((CHUNK, D), jnp.float32),
        pltpu.VMEM((CHUNK, D), jnp.float32),
        pltpu.VMEM((CHUNK, D), jnp.float32),
        pltpu.VMEM((CHUNK, D), jnp.float32),
        pltpu.VMEM((CHUNK, D), jnp.float32),
        pltpu.SemaphoreType.DMA,
        pltpu.SemaphoreType.DMA,
        pltpu.SemaphoreType.DMA,
        pltpu.SemaphoreType.DMA,
        pltpu.SemaphoreType.DMA,
        pltpu.SemaphoreType.DMA,
    ],
)


_BN = 512
_BE = 2560


def _diag(v_row):
    ri = lax.broadcasted_iota(jnp.int32, (_BN, _BN), 0)
    ci = lax.broadcasted_iota(jnp.int32, (_BN, _BN), 1)
    eye = jnp.where(ri == ci, 1.0, 0.0).astype(jnp.float32)
    return eye * v_row


def _tc1_body(hist_ref, x_ref, w1_ref, y1_ref, disf_ref):
    deg = jnp.sum(hist_ref[...], axis=0, keepdims=True) + 1.0
    disd = _diag(lax.rsqrt(deg))
    xw = jnp.dot(x_ref[...], w1_ref[...], preferred_element_type=jnp.float32)
    y1_ref[...] = jnp.dot(disd, xw, preferred_element_type=jnp.float32)
    disf_ref[...] = jnp.dot(disd, jnp.ones((_BN, D), jnp.float32),
                            preferred_element_type=jnp.float32)


def _tc2_body(sa_ref, sb_ref, y1_ref, dis_ref, b1_ref, w2_ref, y2_ref):
    dis = dis_ref[...]
    h1 = jnp.maximum(
        dis * (sa_ref[...] + sb_ref[...] + y1_ref[...]) + b1_ref[...], 0.0)
    y2_ref[...] = jnp.dot(h1, w2_ref[...],
                          preferred_element_type=jnp.float32) * dis


def _tc3_body(sa_ref, sb_ref, y2_ref, dis_ref, b2_ref, la_ref, lb_ref,
              a_ref, bm_ref):
    h2 = (dis_ref[...] * (sa_ref[...] + sb_ref[...] + y2_ref[...])
          + b2_ref[...])
    a_ref[...] = jnp.dot(h2, la_ref[...], preferred_element_type=jnp.float32)
    bm_ref[...] = jnp.dot(h2, lb_ref[...], preferred_element_type=jnp.float32)


def _tc4_body(g_ref, attrt_ref, lc_ref, lb1_ref, w_ref, b_ref, ot_ref):
    p = lax.dot_general(attrt_ref[...], lc_ref[...],
                        (((0,), (0,)), ((), ())),
                        preferred_element_type=jnp.float32)
    e = jnp.maximum(g_ref[...] + p + lb1_ref[...], 0.0)
    ot_ref[...] = lax.dot_general(
        w_ref[...], e, (((0,), (1,)), ((), ())),
        preferred_element_type=jnp.float32) + b_ref[...]


def _node_spec():
    return pl.BlockSpec((_BN, D), lambda i: (i, 0))


def _full(shape):
    return pl.BlockSpec(shape, lambda i: tuple(0 for _ in shape))


def kernel(x, edge_index, edge_attr, W1, b1, W2, b2, lW1, lb1, lW2, lb2):
    f32 = jnp.float32
    src = edge_index[0]
    dst = edge_index[1]
    dst2 = dst.reshape(NW, NCHUNK, CHUNK)

    hist = _deg_call(dst)
    hist2 = hist.reshape(NW, N)

    grid_n = pl.cdiv(N, _BN)
    y1, disf = pl.pallas_call(
        _tc1_body,
        grid=(grid_n,),
        in_specs=[pl.BlockSpec((NW, _BN), lambda i: (0, i)),
                  _node_spec(), _full((D, H))],
        out_specs=[_node_spec(), _node_spec()],
        out_shape=[jax.ShapeDtypeStruct((N, H), f32),
                   jax.ShapeDtypeStruct((N, H), f32)],
    )(hist2, x, W1)

    s1 = _seg_call(y1, src, dst2)
    _sa = pl.BlockSpec((_BN, D), lambda i: (i, 0))
    _sb = pl.BlockSpec((_BN, D), lambda i: (i + N16 // _BN, 0))

    y2 = pl.pallas_call(
        _tc2_body,
        grid=(grid_n,),
        in_specs=[_sa, _sb, _node_spec(), _node_spec(),
                  _full((1, H)), _full((H, H))],
        out_specs=_node_spec(),
        out_shape=jax.ShapeDtypeStruct((N, H), f32),
    )(s1, s1, y1, disf, b1.reshape(1, H), W2)

    s2 = _seg_call(y2, src, dst2)

    a_n, b_n = pl.pallas_call(
        _tc3_body,
        grid=(grid_n,),
        in_specs=[_sa, _sb, _node_spec(), _node_spec(),
                  _full((1, H)), _full((H, H)), _full((H, H))],
        out_specs=[_node_spec(), _node_spec()],
        out_shape=[jax.ShapeDtypeStruct((N, H), f32),
                   jax.ShapeDtypeStruct((N, H), f32)],
    )(s2, s2, y2, disf, b2.reshape(1, H), lW1[:H], lW1[H:2 * H])

    z = _edge_call(a_n, b_n, src, dst)

    out_t = pl.pallas_call(
        _tc4_body,
        grid=(E // _BE,),
        in_specs=[pl.BlockSpec((_BE, H), lambda i: (i, 0)),
                  pl.BlockSpec((DE, _BE), lambda i: (0, i)),
                  _full((DE, H)), _full((1, H)), _full((H, C)),
                  _full((C, 1))],
        out_specs=pl.BlockSpec((C, _BE), lambda i: (0, i)),
        out_shape=jax.ShapeDtypeStruct((C, E), f32),
    )(z, edge_attr.T, lW1[2 * H:], lb1.reshape(1, H), lW2,
      lb2.reshape(C, 1))
    return out_t.T


# speedup vs baseline: 16.9343x; 1.0017x over previous
import jax
import jax.numpy as jnp
from jax import lax
from jax.experimental import pallas as pl
from jax.experimental.pallas import tpu as pltpu
from jax.experimental.pallas import tpu_sc as plsc

N = 10000
E = 320000
D = 128
H = 128
DE = 16
C = 2

NC = 2
NS = 16
NW = NC * NS
EPW = E // NW
CHUNK = 80
NCHUNK = EPW // CHUNK
N16 = 10240
RPS = N16 // NS

_MESH = plsc.VectorSubcoreMesh(
    core_axis_name="c", subcore_axis_name="s", num_cores=NC, num_subcores=NS)
_SC_PARAMS = pltpu.CompilerParams(needs_layout_passes=False)


def _wid():
    return lax.axis_index("s") * NC + lax.axis_index("c")


def _deg_body(dst_hbm, hist_hbm, dst_v, hist_v):
    w = _wid()
    pltpu.sync_copy(dst_hbm.at[pl.ds(w * EPW, EPW)], dst_v)
    zeros = jnp.zeros((16,), jnp.float32)

    def _z(i, _):
        hist_v[pl.ds(i * 16, 16)] = zeros
        return 0
    lax.fori_loop(0, N // 16, _z, 0, unroll=8)

    ones = jnp.full((16,), 1.0, jnp.float32)

    def _acc(i, _):
        idx = dst_v[pl.ds(i * 16, 16)]
        plsc.addupdate_scatter(hist_v, [idx], ones)
        return 0
    lax.fori_loop(0, EPW // 16, _acc, 0, unroll=8)
    pltpu.sync_copy(hist_v, hist_hbm.at[pl.ds(w * N, N)])


_deg_call = pl.kernel(
    _deg_body,
    out_type=jax.ShapeDtypeStruct((NW * N,), jnp.float32),
    mesh=_MESH,
    compiler_params=_SC_PARAMS,
    scratch_types=[
        pltpu.VMEM((EPW,), jnp.int32),
        pltpu.VMEM((N,), jnp.float32),
    ],
)


def _seg_body(y_hbm, src_hbm, dst_hbm, out_hbm,
              src_v, dst_v, rows0, rows1, zbuf, acc, sem0, sem1):
    w = _wid()
    sid = lax.axis_index("s")
    cid = lax.axis_index("c")
    pltpu.sync_copy(src_hbm.at[pl.ds(w * EPW, EPW)], src_v)
    pltpu.sync_copy(dst_hbm.at[w], dst_v)

    zeros = jnp.zeros((16,), jnp.float32)

    def _z(i, _):
        zbuf[i // 8, pl.ds((i % 8) * 16, 16)] = zeros
        return 0
    lax.fori_loop(0, 64, _z, 0, unroll=8)

    def _zc(k, _):
        pltpu.sync_copy(zbuf, acc.at[pl.ds(sid * RPS + k * 8, 8)])
        return 0
    lax.fori_loop(0, RPS // 8, _zc, 0)
    plsc.subcore_barrier()

    def _sl(j):
        return src_v.at[pl.ds(j * CHUNK, CHUNK)]

    pltpu.async_copy(y_hbm.at[_sl(0)], rows0, sem0)

    def _step(j, _):
        even = j % 2 == 0

        @pl.when(j + 1 < NCHUNK)
        def _():
            @pl.when(even)
            def _():
                pltpu.async_copy(y_hbm.at[_sl(j + 1)], rows1, sem1)

            @pl.when(jnp.logical_not(even))
            def _():
                pltpu.async_copy(y_hbm.at[_sl(j + 1)], rows0, sem0)

        @pl.when(even)
        def _():
            pltpu.make_async_copy(y_hbm.at[_sl(j)], rows0, sem0).wait()
            pltpu.sync_copy(rows0, acc.at[dst_v.at[j]], add=True)

        @pl.when(jnp.logical_not(even))
        def _():
            pltpu.make_async_copy(y_hbm.at[_sl(j)], rows1, sem1).wait()
            pltpu.sync_copy(rows1, acc.at[dst_v.at[j]], add=True)
        return 0

    lax.fori_loop(0, NCHUNK, _step, 0)
    plsc.subcore_barrier()
    pltpu.sync_copy(acc.at[pl.ds(sid * RPS, RPS)],
                    out_hbm.at[pl.ds(cid * N16 + sid * RPS, RPS)])


_seg_call = pl.kernel(
    _seg_body,
    out_type=jax.ShapeDtypeStruct((NC * N16, D), jnp.float32),
    mesh=_MESH,
    compiler_params=_SC_PARAMS,
    scratch_types=[
        pltpu.VMEM((EPW,), jnp.int32),
        pltpu.VMEM((NCHUNK, CHUNK), jnp.int32),
        pltpu.VMEM((CHUNK, D), jnp.float32),
        pltpu.VMEM((CHUNK, D), jnp.float32),
        pltpu.VMEM((8, D), jnp.float32),
        pltpu.VMEM_SHARED((N16, D), jnp.float32),
        pltpu.SemaphoreType.DMA,
        pltpu.SemaphoreType.DMA,
    ],
)


def _make_edge(ne, ck):
    epw = ne // NW
    nchunk = epw // ck

    def _edge_body(a_hbm, b_hbm, src_hbm, dst_hbm, z_hbm,
                   src_v, dst_v, a0, a1, b0, b1, o0, o1,
                   sa0, sa1, sb0, sb1, so0, so1):
        w = _wid()
        base = w * epw
        pltpu.sync_copy(src_hbm.at[pl.ds(base, epw)], src_v)
        pltpu.sync_copy(dst_hbm.at[pl.ds(base, epw)], dst_v)

        def _sl(v, j):
            return v.at[pl.ds(j * ck, ck)]

        def _zsl(j):
            return z_hbm.at[pl.ds(base + j * ck, ck)]

        pltpu.async_copy(a_hbm.at[_sl(src_v, 0)], a0, sa0)
        pltpu.async_copy(b_hbm.at[_sl(dst_v, 0)], b0, sb0)

        def _wait_store(obuf, osem):
            pltpu.make_async_copy(obuf, z_hbm.at[pl.ds(base, ck)],
                                  osem).wait()

        def _proc(j, abuf, bbuf, obuf, asem, bsem, osem):
            @pl.when(j >= 2)
            def _():
                _wait_store(obuf, osem)
            pltpu.make_async_copy(a_hbm.at[_sl(src_v, j)], abuf, asem).wait()
            pltpu.make_async_copy(b_hbm.at[_sl(dst_v, j)], bbuf, bsem).wait()

            @plsc.parallel_loop(0, ck, step=1, unroll=8)
            def _c(r):
                for c in range(8):
                    col = pl.ds(c * 16, 16)
                    obuf[r, col] = abuf[r, col] + bbuf[r, col]
            pltpu.async_copy(obuf, _zsl(j), osem)

        def _step(j, _):
            even = j % 2 == 0

            @pl.when(jnp.logical_and(even, j + 1 < nchunk))
            def _():
                pltpu.async_copy(a_hbm.at[_sl(src_v, j + 1)], a1, sa1)
                pltpu.async_copy(b_hbm.at[_sl(dst_v, j + 1)], b1, sb1)

            @pl.when(jnp.logical_and(jnp.logical_not(even), j + 1 < nchunk))
            def _():
                pltpu.async_copy(a_hbm.at[_sl(src_v, j + 1)], a0, sa0)
                pltpu.async_copy(b_hbm.at[_sl(dst_v, j + 1)], b0, sb0)

            @pl.when(even)
            def _():
                _proc(j, a0, b0, o0, sa0, sb0, so0)

            @pl.when(jnp.logical_not(even))
            def _():
                _proc(j, a1, b1, o1, sa1, sb1, so1)
            return 0

        lax.fori_loop(0, nchunk, _step, 0)
        _wait_store(o0, so0)
        _wait_store(o1, so1)

    return pl.kernel(
        _edge_body,
        out_type=jax.ShapeDtypeStruct((ne, D), jnp.float32),
        mesh=_MESH,
        compiler_params=_SC_PARAMS,
        scratch_types=[
            pltpu.VMEM((epw,), jnp.int32),
            pltpu.VMEM((epw,), jnp.int32),
            pltpu.VMEM((ck, D), jnp.float32),
            pltpu.VMEM((ck, D), jnp.float32),
            pltpu.VMEM((ck, D), jnp.float32),
            pltpu.VMEM((ck, D), jnp.float32),
            pltpu.VMEM((ck, D), jnp.float32),
            pltpu.VMEM((ck, D), jnp.float32),
            pltpu.SemaphoreType.DMA,
            pltpu.SemaphoreType.DMA,
            pltpu.SemaphoreType.DMA,
            pltpu.SemaphoreType.DMA,
            pltpu.SemaphoreType.DMA,
            pltpu.SemaphoreType.DMA,
        ],
    )


_EH = E // 2
_edge_call_half = _make_edge(_EH, 40)


_BN = 512
_BE = 3200


def _diag(v_row):
    ri = lax.broadcasted_iota(jnp.int32, (_BN, _BN), 0)
    ci = lax.broadcasted_iota(jnp.int32, (_BN, _BN), 1)
    eye = jnp.where(ri == ci, 1.0, 0.0).astype(jnp.float32)
    return eye * v_row


def _tc1_body(hist_ref, x_ref, w1_ref, y1_ref, disf_ref):
    deg = jnp.sum(hist_ref[...], axis=0, keepdims=True) + 1.0
    disd = _diag(lax.rsqrt(deg))
    xw = jnp.dot(x_ref[...], w1_ref[...], preferred_element_type=jnp.float32)
    y1_ref[...] = jnp.dot(disd, xw, preferred_element_type=jnp.float32)
    disf_ref[...] = jnp.dot(disd, jnp.ones((_BN, D), jnp.float32),
                            preferred_element_type=jnp.float32)


def _tc2_body(sa_ref, sb_ref, y1_ref, dis_ref, b1_ref, w2_ref, y2_ref):
    dis = dis_ref[...]
    h1 = jnp.maximum(
        dis * (sa_ref[...] + sb_ref[...] + y1_ref[...]) + b1_ref[...], 0.0)
    y2_ref[...] = jnp.dot(h1, w2_ref[...],
                          preferred_element_type=jnp.float32) * dis


def _tc3_body(sa_ref, sb_ref, y2_ref, dis_ref, b2_ref, la_ref, lb_ref,
              a_ref, bm_ref):
    h2 = (dis_ref[...] * (sa_ref[...] + sb_ref[...] + y2_ref[...])
          + b2_ref[...])
    a_ref[...] = jnp.dot(h2, la_ref[...], preferred_element_type=jnp.float32)
    bm_ref[...] = jnp.dot(h2, lb_ref[...], preferred_element_type=jnp.float32)


def _tc4_body(g_ref, attrt_ref, lc_ref, lb1_ref, w_ref, b_ref, ot_ref):
    p = lax.dot_general(attrt_ref[...], lc_ref[...],
                        (((0,), (0,)), ((), ())),
                        preferred_element_type=jnp.float32)
    e = jnp.maximum(g_ref[...] + p + lb1_ref[...], 0.0)
    ot_ref[...] = lax.dot_general(
        w_ref[...], e, (((0,), (1,)), ((), ())),
        preferred_element_type=jnp.float32) + b_ref[...]


def _node_spec():
    return pl.BlockSpec((_BN, D), lambda i: (i, 0))


def _full(shape):
    return pl.BlockSpec(shape, lambda i: tuple(0 for _ in shape))


def kernel(x, edge_index, edge_attr, W1, b1, W2, b2, lW1, lb1, lW2, lb2):
    f32 = jnp.float32
    src = edge_index[0]
    dst = edge_index[1]
    dst2 = dst.reshape(NW, NCHUNK, CHUNK)

    hist = _deg_call(dst)
    hist2 = hist.reshape(NW, N)

    grid_n = pl.cdiv(N, _BN)
    y1, disf = pl.pallas_call(
        _tc1_body,
        grid=(grid_n,),
        in_specs=[pl.BlockSpec((NW, _BN), lambda i: (0, i)),
                  _node_spec(), _full((D, H))],
        out_specs=[_node_spec(), _node_spec()],
        out_shape=[jax.ShapeDtypeStruct((N, H), f32),
                   jax.ShapeDtypeStruct((N, H), f32)],
    )(hist2, x, W1)

    s1 = _seg_call(y1, src, dst2)
    _sa = pl.BlockSpec((_BN, D), lambda i: (i, 0))
    _sb = pl.BlockSpec((_BN, D), lambda i: (i + N16 // _BN, 0))

    y2 = pl.pallas_call(
        _tc2_body,
        grid=(grid_n,),
        in_specs=[_sa, _sb, _node_spec(), _node_spec(),
                  _full((1, H)), _full((H, H))],
        out_specs=_node_spec(),
        out_shape=jax.ShapeDtypeStruct((N, H), f32),
    )(s1, s1, y1, disf, b1.reshape(1, H), W2)

    s2 = _seg_call(y2, src, dst2)

    a_n, b_n = pl.pallas_call(
        _tc3_body,
        grid=(grid_n,),
        in_specs=[_sa, _sb, _node_spec(), _node_spec(),
                  _full((1, H)), _full((H, H)), _full((H, H))],
        out_specs=[_node_spec(), _node_spec()],
        out_shape=[jax.ShapeDtypeStruct((N, H), f32),
                   jax.ShapeDtypeStruct((N, H), f32)],
    )(s2, s2, y2, disf, b2.reshape(1, H), lW1[:H], lW1[H:2 * H])

    attr_t = edge_attr.T
    lc = lW1[2 * H:]
    lb1r = lb1.reshape(1, H)
    lb2r = lb2.reshape(C, 1)

    def _tc4(zh, ath):
        return pl.pallas_call(
            _tc4_body,
            grid=(_EH // _BE,),
            in_specs=[pl.BlockSpec((_BE, H), lambda i: (i, 0)),
                      pl.BlockSpec((DE, _BE), lambda i: (0, i)),
                      _full((DE, H)), _full((1, H)), _full((H, C)),
                      _full((C, 1))],
            out_specs=pl.BlockSpec((C, _BE), lambda i: (0, i)),
            out_shape=jax.ShapeDtypeStruct((C, _EH), f32),
        )(zh, ath, lc, lb1r, lW2, lb2r)

    z_a = _edge_call_half(a_n, b_n, src[:_EH], dst[:_EH])
    z_b = _edge_call_half(a_n, b_n, src[_EH:], dst[_EH:])
    out_a = _tc4(z_a, attr_t[:, :_EH])
    out_b = _tc4(z_b, attr_t[:, _EH:])
    return jnp.concatenate([out_a, out_b], axis=1).T


# speedup vs baseline: 17.3053x; 1.0219x over previous
import jax
import jax.numpy as jnp
from jax import lax
from jax.experimental import pallas as pl
from jax.experimental.pallas import tpu as pltpu
from jax.experimental.pallas import tpu_sc as plsc

N = 10000
E = 320000
D = 128
H = 128
DE = 16
C = 2

NC = 2
NS = 16
NW = NC * NS
EPW = E // NW
CHUNK = 80
NCHUNK = EPW // CHUNK
N16 = 10240
RPS = N16 // NS

_MESH = plsc.VectorSubcoreMesh(
    core_axis_name="c", subcore_axis_name="s", num_cores=NC, num_subcores=NS)
_SC_PARAMS = pltpu.CompilerParams(needs_layout_passes=False)


def _wid():
    return lax.axis_index("s") * NC + lax.axis_index("c")


def _deg_body(dst_hbm, hist_hbm, dst_v, hist_v):
    w = _wid()
    pltpu.sync_copy(dst_hbm.at[pl.ds(w * EPW, EPW)], dst_v)
    zeros = jnp.zeros((16,), jnp.float32)

    def _z(i, _):
        hist_v[pl.ds(i * 16, 16)] = zeros
        return 0
    lax.fori_loop(0, N // 16, _z, 0, unroll=8)

    ones = jnp.full((16,), 1.0, jnp.float32)

    def _acc(i, _):
        idx = dst_v[pl.ds(i * 16, 16)]
        plsc.addupdate_scatter(hist_v, [idx], ones)
        return 0
    lax.fori_loop(0, EPW // 16, _acc, 0, unroll=8)
    pltpu.sync_copy(hist_v, hist_hbm.at[pl.ds(w * N, N)])


_deg_call = pl.kernel(
    _deg_body,
    out_type=jax.ShapeDtypeStruct((NW * N,), jnp.float32),
    mesh=_MESH,
    compiler_params=_SC_PARAMS,
    scratch_types=[
        pltpu.VMEM((EPW,), jnp.int32),
        pltpu.VMEM((N,), jnp.float32),
    ],
)


def _seg_body(y_hbm, src_hbm, dst_hbm, out_hbm,
              src_v, dst_v, rows0, rows1, zbuf, acc, sem0, sem1):
    w = _wid()
    sid = lax.axis_index("s")
    cid = lax.axis_index("c")
    pltpu.sync_copy(src_hbm.at[pl.ds(w * EPW, EPW)], src_v)
    pltpu.sync_copy(dst_hbm.at[w], dst_v)

    zeros = jnp.zeros((16,), jnp.float32)

    def _z(i, _):
        zbuf[i // 8, pl.ds((i % 8) * 16, 16)] = zeros
        return 0
    lax.fori_loop(0, 64, _z, 0, unroll=8)

    def _zc(k, _):
        pltpu.sync_copy(zbuf, acc.at[pl.ds(sid * RPS + k * 8, 8)])
        return 0
    lax.fori_loop(0, RPS // 8, _zc, 0)
    plsc.subcore_barrier()

    def _sl(j):
        return src_v.at[pl.ds(j * CHUNK, CHUNK)]

    pltpu.async_copy(y_hbm.at[_sl(0)], rows0, sem0)

    def _step(j, _):
        even = j % 2 == 0

        @pl.when(j + 1 < NCHUNK)
        def _():
            @pl.when(even)
            def _():
                pltpu.async_copy(y_hbm.at[_sl(j + 1)], rows1, sem1)

            @pl.when(jnp.logical_not(even))
            def _():
                pltpu.async_copy(y_hbm.at[_sl(j + 1)], rows0, sem0)

        @pl.when(even)
        def _():
            pltpu.make_async_copy(y_hbm.at[_sl(j)], rows0, sem0).wait()
            pltpu.sync_copy(rows0, acc.at[dst_v.at[j]], add=True)

        @pl.when(jnp.logical_not(even))
        def _():
            pltpu.make_async_copy(y_hbm.at[_sl(j)], rows1, sem1).wait()
            pltpu.sync_copy(rows1, acc.at[dst_v.at[j]], add=True)
        return 0

    lax.fori_loop(0, NCHUNK, _step, 0)
    plsc.subcore_barrier()
    pltpu.sync_copy(acc.at[pl.ds(sid * RPS, RPS)],
                    out_hbm.at[pl.ds(cid * N16 + sid * RPS, RPS)])


_seg_call = pl.kernel(
    _seg_body,
    out_type=jax.ShapeDtypeStruct((NC * N16, D), jnp.float32),
    mesh=_MESH,
    compiler_params=_SC_PARAMS,
    scratch_types=[
        pltpu.VMEM((EPW,), jnp.int32),
        pltpu.VMEM((NCHUNK, CHUNK), jnp.int32),
        pltpu.VMEM((CHUNK, D), jnp.float32),
        pltpu.VMEM((CHUNK, D), jnp.float32),
        pltpu.VMEM((8, D), jnp.float32),
        pltpu.VMEM_SHARED((N16, D), jnp.float32),
        pltpu.SemaphoreType.DMA,
        pltpu.SemaphoreType.DMA,
    ],
)


def _make_edge(ne, ck):
    epw = ne // NW
    nchunk = epw // ck

    def _edge_body(a_hbm, b_hbm, src_hbm, dst_hbm, z_hbm,
                   src_v, dst_v, a0, a1, b0, b1, o0, o1,
                   sa0, sa1, sb0, sb1, so0, so1):
        w = _wid()
        base = w * epw
        pltpu.sync_copy(src_hbm.at[pl.ds(base, epw)], src_v)
        pltpu.sync_copy(dst_hbm.at[pl.ds(base, epw)], dst_v)

        def _sl(v, j):
            return v.at[pl.ds(j * ck, ck)]

        def _zsl(j):
            return z_hbm.at[pl.ds(base + j * ck, ck)]

        pltpu.async_copy(a_hbm.at[_sl(src_v, 0)], a0, sa0)
        pltpu.async_copy(b_hbm.at[_sl(dst_v, 0)], b0, sb0)

        def _wait_store(obuf, osem):
            pltpu.make_async_copy(obuf, z_hbm.at[pl.ds(base, ck)],
                                  osem).wait()

        def _proc(j, abuf, bbuf, obuf, asem, bsem, osem):
            @pl.when(j >= 2)
            def _():
                _wait_store(obuf, osem)
            pltpu.make_async_copy(a_hbm.at[_sl(src_v, j)], abuf, asem).wait()
            pltpu.make_async_copy(b_hbm.at[_sl(dst_v, j)], bbuf, bsem).wait()

            @plsc.parallel_loop(0, ck, step=1, unroll=8)
            def _c(r):
                for c in range(8):
                    col = pl.ds(c * 16, 16)
                    obuf[r, col] = abuf[r, col] + bbuf[r, col]
            pltpu.async_copy(obuf, _zsl(j), osem)

        def _step(j, _):
            even = j % 2 == 0

            @pl.when(jnp.logical_and(even, j + 1 < nchunk))
            def _():
                pltpu.async_copy(a_hbm.at[_sl(src_v, j + 1)], a1, sa1)
                pltpu.async_copy(b_hbm.at[_sl(dst_v, j + 1)], b1, sb1)

            @pl.when(jnp.logical_and(jnp.logical_not(even), j + 1 < nchunk))
            def _():
                pltpu.async_copy(a_hbm.at[_sl(src_v, j + 1)], a0, sa0)
                pltpu.async_copy(b_hbm.at[_sl(dst_v, j + 1)], b0, sb0)

            @pl.when(even)
            def _():
                _proc(j, a0, b0, o0, sa0, sb0, so0)

            @pl.when(jnp.logical_not(even))
            def _():
                _proc(j, a1, b1, o1, sa1, sb1, so1)
            return 0

        lax.fori_loop(0, nchunk, _step, 0)
        _wait_store(o0, so0)
        _wait_store(o1, so1)

    return pl.kernel(
        _edge_body,
        out_type=jax.ShapeDtypeStruct((ne, D), jnp.float32),
        mesh=_MESH,
        compiler_params=_SC_PARAMS,
        scratch_types=[
            pltpu.VMEM((epw,), jnp.int32),
            pltpu.VMEM((epw,), jnp.int32),
            pltpu.VMEM((ck, D), jnp.float32),
            pltpu.VMEM((ck, D), jnp.float32),
            pltpu.VMEM((ck, D), jnp.float32),
            pltpu.VMEM((ck, D), jnp.float32),
            pltpu.VMEM((ck, D), jnp.float32),
            pltpu.VMEM((ck, D), jnp.float32),
            pltpu.SemaphoreType.DMA,
            pltpu.SemaphoreType.DMA,
            pltpu.SemaphoreType.DMA,
            pltpu.SemaphoreType.DMA,
            pltpu.SemaphoreType.DMA,
            pltpu.SemaphoreType.DMA,
        ],
    )


_EA = 192000
_EB = E - _EA
_edge_call_a = _make_edge(_EA, CHUNK)
_edge_call_b = _make_edge(_EB, CHUNK)


_BN = 512
_BE = 3200


def _diag(v_row):
    ri = lax.broadcasted_iota(jnp.int32, (_BN, _BN), 0)
    ci = lax.broadcasted_iota(jnp.int32, (_BN, _BN), 1)
    eye = jnp.where(ri == ci, 1.0, 0.0).astype(jnp.float32)
    return eye * v_row


def _tc1_body(hist_ref, x_ref, w1_ref, y1_ref, disf_ref):
    deg = jnp.sum(hist_ref[...], axis=0, keepdims=True) + 1.0
    disd = _diag(lax.rsqrt(deg))
    xw = jnp.dot(x_ref[...], w1_ref[...], preferred_element_type=jnp.float32)
    y1_ref[...] = jnp.dot(disd, xw, preferred_element_type=jnp.float32)
    disf_ref[...] = jnp.dot(disd, jnp.ones((_BN, D), jnp.float32),
                            preferred_element_type=jnp.float32)


def _tc2_body(sa_ref, sb_ref, y1_ref, dis_ref, b1_ref, w2_ref, y2_ref):
    dis = dis_ref[...]
    h1 = jnp.maximum(
        dis * (sa_ref[...] + sb_ref[...] + y1_ref[...]) + b1_ref[...], 0.0)
    y2_ref[...] = jnp.dot(h1, w2_ref[...],
                          preferred_element_type=jnp.float32) * dis


def _tc3_body(sa_ref, sb_ref, y2_ref, dis_ref, b2_ref, la_ref, lb_ref,
              a_ref, bm_ref):
    h2 = (dis_ref[...] * (sa_ref[...] + sb_ref[...] + y2_ref[...])
          + b2_ref[...])
    a_ref[...] = jnp.dot(h2, la_ref[...], preferred_element_type=jnp.float32)
    bm_ref[...] = jnp.dot(h2, lb_ref[...], preferred_element_type=jnp.float32)


def _tc4_body(g_ref, attrt_ref, lc_ref, lb1_ref, w_ref, b_ref, ot_ref):
    p = lax.dot_general(attrt_ref[...], lc_ref[...],
                        (((0,), (0,)), ((), ())),
                        preferred_element_type=jnp.float32)
    e = jnp.maximum(g_ref[...] + p + lb1_ref[...], 0.0)
    ot_ref[...] = lax.dot_general(
        w_ref[...], e, (((0,), (1,)), ((), ())),
        preferred_element_type=jnp.float32) + b_ref[...]


def _node_spec():
    return pl.BlockSpec((_BN, D), lambda i: (i, 0))


def _full(shape):
    return pl.BlockSpec(shape, lambda i: tuple(0 for _ in shape))


def kernel(x, edge_index, edge_attr, W1, b1, W2, b2, lW1, lb1, lW2, lb2):
    f32 = jnp.float32
    src = edge_index[0]
    dst = edge_index[1]
    dst2 = dst.reshape(NW, NCHUNK, CHUNK)

    hist = _deg_call(dst)
    hist2 = hist.reshape(NW, N)

    grid_n = pl.cdiv(N, _BN)
    y1, disf = pl.pallas_call(
        _tc1_body,
        grid=(grid_n,),
        in_specs=[pl.BlockSpec((NW, _BN), lambda i: (0, i)),
                  _node_spec(), _full((D, H))],
        out_specs=[_node_spec(), _node_spec()],
        out_shape=[jax.ShapeDtypeStruct((N, H), f32),
                   jax.ShapeDtypeStruct((N, H), f32)],
    )(hist2, x, W1)

    s1 = _seg_call(y1, src, dst2)
    _sa = pl.BlockSpec((_BN, D), lambda i: (i, 0))
    _sb = pl.BlockSpec((_BN, D), lambda i: (i + N16 // _BN, 0))

    y2 = pl.pallas_call(
        _tc2_body,
        grid=(grid_n,),
        in_specs=[_sa, _sb, _node_spec(), _node_spec(),
                  _full((1, H)), _full((H, H))],
        out_specs=_node_spec(),
        out_shape=jax.ShapeDtypeStruct((N, H), f32),
    )(s1, s1, y1, disf, b1.reshape(1, H), W2)

    s2 = _seg_call(y2, src, dst2)

    a_n, b_n = pl.pallas_call(
        _tc3_body,
        grid=(grid_n,),
        in_specs=[_sa, _sb, _node_spec(), _node_spec(),
                  _full((1, H)), _full((H, H)), _full((H, H))],
        out_specs=[_node_spec(), _node_spec()],
        out_shape=[jax.ShapeDtypeStruct((N, H), f32),
                   jax.ShapeDtypeStruct((N, H), f32)],
    )(s2, s2, y2, disf, b2.reshape(1, H), lW1[:H], lW1[H:2 * H])

    attr_t = edge_attr.T
    lc = lW1[2 * H:]
    lb1r = lb1.reshape(1, H)
    lb2r = lb2.reshape(C, 1)

    def _tc4(zh, ath, ne):
        return pl.pallas_call(
            _tc4_body,
            grid=(ne // _BE,),
            in_specs=[pl.BlockSpec((_BE, H), lambda i: (i, 0)),
                      pl.BlockSpec((DE, _BE), lambda i: (0, i)),
                      _full((DE, H)), _full((1, H)), _full((H, C)),
                      _full((C, 1))],
            out_specs=pl.BlockSpec((C, _BE), lambda i: (0, i)),
            out_shape=jax.ShapeDtypeStruct((C, ne), f32),
        )(zh, ath, lc, lb1r, lW2, lb2r)

    z_a = _edge_call_a(a_n, b_n, src[:_EA], dst[:_EA])
    z_b = _edge_call_b(a_n, b_n, src[_EA:], dst[_EA:])
    out_a = _tc4(z_a, attr_t[:, :_EA], _EA)
    out_b = _tc4(z_b, attr_t[:, _EA:], _EB)
    return jnp.concatenate([out_a, out_b], axis=1).T


# speedup vs baseline: 17.3689x; 1.0037x over previous
import jax
import jax.numpy as jnp
from jax import lax
from jax.experimental import pallas as pl
from jax.experimental.pallas import tpu as pltpu
from jax.experimental.pallas import tpu_sc as plsc

N = 10000
E = 320000
D = 128
H = 128
DE = 16
C = 2

NC = 2
NS = 16
NW = NC * NS
EPW = E // NW
CHUNK = 80
NCHUNK = EPW // CHUNK
N16 = 10240
RPS = N16 // NS

_MESH = plsc.VectorSubcoreMesh(
    core_axis_name="c", subcore_axis_name="s", num_cores=NC, num_subcores=NS)
_SC_PARAMS = pltpu.CompilerParams(needs_layout_passes=False)


def _wid():
    return lax.axis_index("s") * NC + lax.axis_index("c")


def _deg_body(dst_hbm, hist_hbm, dst_v, hist_v):
    w = _wid()
    pltpu.sync_copy(dst_hbm.at[pl.ds(w * EPW, EPW)], dst_v)
    zeros = jnp.zeros((16,), jnp.float32)

    def _z(i, _):
        hist_v[pl.ds(i * 16, 16)] = zeros
        return 0
    lax.fori_loop(0, N // 16, _z, 0, unroll=8)

    ones = jnp.full((16,), 1.0, jnp.float32)

    def _acc(i, _):
        idx = dst_v[pl.ds(i * 16, 16)]
        plsc.addupdate_scatter(hist_v, [idx], ones)
        return 0
    lax.fori_loop(0, EPW // 16, _acc, 0, unroll=8)
    pltpu.sync_copy(hist_v, hist_hbm.at[pl.ds(w * N, N)])


_deg_call = pl.kernel(
    _deg_body,
    out_type=jax.ShapeDtypeStruct((NW * N,), jnp.float32),
    mesh=_MESH,
    compiler_params=_SC_PARAMS,
    scratch_types=[
        pltpu.VMEM((EPW,), jnp.int32),
        pltpu.VMEM((N,), jnp.float32),
    ],
)


def _seg_body(y_hbm, src_hbm, dst_hbm, out_hbm,
              src_v, dst_v, rows0, rows1, zbuf, acc, sem0, sem1):
    w = _wid()
    sid = lax.axis_index("s")
    cid = lax.axis_index("c")
    pltpu.sync_copy(src_hbm.at[pl.ds(w * EPW, EPW)], src_v)
    pltpu.sync_copy(dst_hbm.at[w], dst_v)

    zeros = jnp.zeros((16,), jnp.float32)

    def _z(i, _):
        zbuf[i // 8, pl.ds((i % 8) * 16, 16)] = zeros
        return 0
    lax.fori_loop(0, 64, _z, 0, unroll=8)

    def _zc(k, _):
        pltpu.sync_copy(zbuf, acc.at[pl.ds(sid * RPS + k * 8, 8)])
        return 0
    lax.fori_loop(0, RPS // 8, _zc, 0)
    plsc.subcore_barrier()

    def _sl(j):
        return src_v.at[pl.ds(j * CHUNK, CHUNK)]

    pltpu.async_copy(y_hbm.at[_sl(0)], rows0, sem0)

    def _step(j, _):
        even = j % 2 == 0

        @pl.when(j + 1 < NCHUNK)
        def _():
            @pl.when(even)
            def _():
                pltpu.async_copy(y_hbm.at[_sl(j + 1)], rows1, sem1)

            @pl.when(jnp.logical_not(even))
            def _():
                pltpu.async_copy(y_hbm.at[_sl(j + 1)], rows0, sem0)

        @pl.when(even)
        def _():
            pltpu.make_async_copy(y_hbm.at[_sl(j)], rows0, sem0).wait()
            pltpu.sync_copy(rows0, acc.at[dst_v.at[j]], add=True)

        @pl.when(jnp.logical_not(even))
        def _():
            pltpu.make_async_copy(y_hbm.at[_sl(j)], rows1, sem1).wait()
            pltpu.sync_copy(rows1, acc.at[dst_v.at[j]], add=True)
        return 0

    lax.fori_loop(0, NCHUNK, _step, 0)
    plsc.subcore_barrier()
    pltpu.sync_copy(acc.at[pl.ds(sid * RPS, RPS)],
                    out_hbm.at[pl.ds(cid * N16 + sid * RPS, RPS)])


_seg_call = pl.kernel(
    _seg_body,
    out_type=jax.ShapeDtypeStruct((NC * N16, D), jnp.float32),
    mesh=_MESH,
    compiler_params=_SC_PARAMS,
    scratch_types=[
        pltpu.VMEM((EPW,), jnp.int32),
        pltpu.VMEM((NCHUNK, CHUNK), jnp.int32),
        pltpu.VMEM((CHUNK, D), jnp.float32),
        pltpu.VMEM((CHUNK, D), jnp.float32),
        pltpu.VMEM((8, D), jnp.float32),
        pltpu.VMEM_SHARED((N16, D), jnp.float32),
        pltpu.SemaphoreType.DMA,
        pltpu.SemaphoreType.DMA,
    ],
)


def _make_edge(ne, ck):
    epw = ne // NW
    nchunk = epw // ck

    def _edge_body(a_hbm, b_hbm, src_hbm, dst_hbm, z_hbm,
                   src_v, dst_v, a0, a1, b0, b1, o0, o1,
                   sa0, sa1, sb0, sb1, so0, so1):
        w = _wid()
        base = w * epw
        pltpu.sync_copy(src_hbm.at[pl.ds(base, epw)], src_v)
        pltpu.sync_copy(dst_hbm.at[pl.ds(base, epw)], dst_v)

        def _sl(v, j):
            return v.at[pl.ds(j * ck, ck)]

        def _zsl(j):
            return z_hbm.at[pl.ds(base + j * ck, ck)]

        pltpu.async_copy(a_hbm.at[_sl(src_v, 0)], a0, sa0)
        pltpu.async_copy(b_hbm.at[_sl(dst_v, 0)], b0, sb0)

        def _wait_store(obuf, osem):
            pltpu.make_async_copy(obuf, z_hbm.at[pl.ds(base, ck)],
                                  osem).wait()

        def _proc(j, abuf, bbuf, obuf, asem, bsem, osem):
            @pl.when(j >= 2)
            def _():
                _wait_store(obuf, osem)
            pltpu.make_async_copy(a_hbm.at[_sl(src_v, j)], abuf, asem).wait()
            pltpu.make_async_copy(b_hbm.at[_sl(dst_v, j)], bbuf, bsem).wait()

            @plsc.parallel_loop(0, ck, step=1, unroll=8)
            def _c(r):
                for c in range(8):
                    col = pl.ds(c * 16, 16)
                    obuf[r, col] = abuf[r, col] + bbuf[r, col]
            pltpu.async_copy(obuf, _zsl(j), osem)

        def _step(j, _):
            even = j % 2 == 0

            @pl.when(jnp.logical_and(even, j + 1 < nchunk))
            def _():
                pltpu.async_copy(a_hbm.at[_sl(src_v, j + 1)], a1, sa1)
                pltpu.async_copy(b_hbm.at[_sl(dst_v, j + 1)], b1, sb1)

            @pl.when(jnp.logical_and(jnp.logical_not(even), j + 1 < nchunk))
            def _():
                pltpu.async_copy(a_hbm.at[_sl(src_v, j + 1)], a0, sa0)
                pltpu.async_copy(b_hbm.at[_sl(dst_v, j + 1)], b0, sb0)

            @pl.when(even)
            def _():
                _proc(j, a0, b0, o0, sa0, sb0, so0)

            @pl.when(jnp.logical_not(even))
            def _():
                _proc(j, a1, b1, o1, sa1, sb1, so1)
            return 0

        lax.fori_loop(0, nchunk, _step, 0)
        _wait_store(o0, so0)
        _wait_store(o1, so1)

    return pl.kernel(
        _edge_body,
        out_type=jax.ShapeDtypeStruct((ne, D), jnp.float32),
        mesh=_MESH,
        compiler_params=_SC_PARAMS,
        scratch_types=[
            pltpu.VMEM((epw,), jnp.int32),
            pltpu.VMEM((epw,), jnp.int32),
            pltpu.VMEM((ck, D), jnp.float32),
            pltpu.VMEM((ck, D), jnp.float32),
            pltpu.VMEM((ck, D), jnp.float32),
            pltpu.VMEM((ck, D), jnp.float32),
            pltpu.VMEM((ck, D), jnp.float32),
            pltpu.VMEM((ck, D), jnp.float32),
            pltpu.SemaphoreType.DMA,
            pltpu.SemaphoreType.DMA,
            pltpu.SemaphoreType.DMA,
            pltpu.SemaphoreType.DMA,
            pltpu.SemaphoreType.DMA,
            pltpu.SemaphoreType.DMA,
        ],
    )


_EA = 192000
_EB = E - _EA
_edge_call_a = _make_edge(_EA, CHUNK)
_edge_call_b = _make_edge(_EB, CHUNK)


_BN = 512
_BE = 3200


def _diag(v_row):
    ri = lax.broadcasted_iota(jnp.int32, (_BN, _BN), 0)
    ci = lax.broadcasted_iota(jnp.int32, (_BN, _BN), 1)
    eye = jnp.where(ri == ci, 1.0, 0.0).astype(jnp.float32)
    return eye * v_row


def _tc1_body(hist_ref, x_ref, w1_ref, y1_ref, disf_ref):
    deg = jnp.sum(hist_ref[...], axis=0, keepdims=True) + 1.0
    disd = _diag(lax.rsqrt(deg))
    xw = jnp.dot(x_ref[...], w1_ref[...], preferred_element_type=jnp.float32)
    disf = jnp.dot(disd, jnp.ones((_BN, D), jnp.float32),
                   preferred_element_type=jnp.float32)
    y1_ref[...] = xw * disf
    disf_ref[...] = disf


def _tc2_body(sa_ref, sb_ref, y1_ref, dis_ref, b1_ref, w2_ref, y2_ref):
    dis = dis_ref[...]
    h1 = jnp.maximum(
        dis * (sa_ref[...] + sb_ref[...] + y1_ref[...]) + b1_ref[...], 0.0)
    y2_ref[...] = jnp.dot(h1, w2_ref[...],
                          preferred_element_type=jnp.float32) * dis


def _tc3_body(sa_ref, sb_ref, y2_ref, dis_ref, b2_ref, la_ref, lb_ref,
              a_ref, bm_ref):
    h2 = (dis_ref[...] * (sa_ref[...] + sb_ref[...] + y2_ref[...])
          + b2_ref[...])
    a_ref[...] = jnp.dot(h2, la_ref[...], preferred_element_type=jnp.float32)
    bm_ref[...] = jnp.dot(h2, lb_ref[...], preferred_element_type=jnp.float32)


def _tc4_body(g_ref, attrt_ref, lc_ref, lb1_ref, w_ref, b_ref, ot_ref):
    p = lax.dot_general(attrt_ref[...], lc_ref[...],
                        (((0,), (0,)), ((), ())),
                        preferred_element_type=jnp.float32)
    e = jnp.maximum(g_ref[...] + p + lb1_ref[...], 0.0)
    ot_ref[...] = lax.dot_general(
        w_ref[...], e, (((0,), (1,)), ((), ())),
        preferred_element_type=jnp.float32) + b_ref[...]


def _node_spec():
    return pl.BlockSpec((_BN, D), lambda i: (i, 0))


def _full(shape):
    return pl.BlockSpec(shape, lambda i: tuple(0 for _ in shape))


def kernel(x, edge_index, edge_attr, W1, b1, W2, b2, lW1, lb1, lW2, lb2):
    f32 = jnp.float32
    src = edge_index[0]
    dst = edge_index[1]
    dst2 = dst.reshape(NW, NCHUNK, CHUNK)

    hist = _deg_call(dst)
    hist2 = hist.reshape(NW, N)

    grid_n = pl.cdiv(N, _BN)
    y1, disf = pl.pallas_call(
        _tc1_body,
        grid=(grid_n,),
        in_specs=[pl.BlockSpec((NW, _BN), lambda i: (0, i)),
                  _node_spec(), _full((D, H))],
        out_specs=[_node_spec(), _node_spec()],
        out_shape=[jax.ShapeDtypeStruct((N, H), f32),
                   jax.ShapeDtypeStruct((N, H), f32)],
    )(hist2, x, W1)

    s1 = _seg_call(y1, src, dst2)
    _sa = pl.BlockSpec((_BN, D), lambda i: (i, 0))
    _sb = pl.BlockSpec((_BN, D), lambda i: (i + N16 // _BN, 0))

    y2 = pl.pallas_call(
        _tc2_body,
        grid=(grid_n,),
        in_specs=[_sa, _sb, _node_spec(), _node_spec(),
                  _full((1, H)), _full((H, H))],
        out_specs=_node_spec(),
        out_shape=jax.ShapeDtypeStruct((N, H), f32),
    )(s1, s1, y1, disf, b1.reshape(1, H), W2)

    s2 = _seg_call(y2, src, dst2)

    a_n, b_n = pl.pallas_call(
        _tc3_body,
        grid=(grid_n,),
        in_specs=[_sa, _sb, _node_spec(), _node_spec(),
                  _full((1, H)), _full((H, H)), _full((H, H))],
        out_specs=[_node_spec(), _node_spec()],
        out_shape=[jax.ShapeDtypeStruct((N, H), f32),
                   jax.ShapeDtypeStruct((N, H), f32)],
    )(s2, s2, y2, disf, b2.reshape(1, H), lW1[:H], lW1[H:2 * H])

    attr_t = edge_attr.T
    lc = lW1[2 * H:]
    lb1r = lb1.reshape(1, H)
    lb2r = lb2.reshape(C, 1)

    def _tc4(zh, ath, ne):
        return pl.pallas_call(
            _tc4_body,
            grid=(ne // _BE,),
            in_specs=[pl.BlockSpec((_BE, H), lambda i: (i, 0)),
                      pl.BlockSpec((DE, _BE), lambda i: (0, i)),
                      _full((DE, H)), _full((1, H)), _full((H, C)),
                      _full((C, 1))],
            out_specs=pl.BlockSpec((C, _BE), lambda i: (0, i)),
            out_shape=jax.ShapeDtypeStruct((C, ne), f32),
        )(zh, ath, lc, lb1r, lW2, lb2r)

    z_a = _edge_call_a(a_n, b_n, src[:_EA], dst[:_EA])
    z_b = _edge_call_b(a_n, b_n, src[_EA:], dst[_EA:])
    out_a = _tc4(z_a, attr_t[:, :_EA], _EA)
    out_b = _tc4(z_b, attr_t[:, _EA:], _EB)
    return jnp.concatenate([out_a, out_b], axis=1).T


# speedup vs baseline: 17.4232x; 1.0031x over previous
import jax
import jax.numpy as jnp
from jax import lax
from jax.experimental import pallas as pl
from jax.experimental.pallas import tpu as pltpu
from jax.experimental.pallas import tpu_sc as plsc

N = 10000
E = 320000
D = 128
H = 128
DE = 16
C = 2

NC = 2
NS = 16
NW = NC * NS
EPW = E // NW
CHUNK = 80
NCHUNK = EPW // CHUNK
N16 = 10240
RPS = N16 // NS

_MESH = plsc.VectorSubcoreMesh(
    core_axis_name="c", subcore_axis_name="s", num_cores=NC, num_subcores=NS)
_SC_PARAMS = pltpu.CompilerParams(needs_layout_passes=False)


def _wid():
    return lax.axis_index("s") * NC + lax.axis_index("c")


def _deg_body(dst_hbm, hist_hbm, dst_v, hist_v):
    w = _wid()
    pltpu.sync_copy(dst_hbm.at[pl.ds(w * EPW, EPW)], dst_v)
    zeros = jnp.zeros((16,), jnp.float32)

    def _z(i, _):
        hist_v[pl.ds(i * 16, 16)] = zeros
        return 0
    lax.fori_loop(0, N // 16, _z, 0, unroll=8)

    ones = jnp.full((16,), 1.0, jnp.float32)

    def _acc(i, _):
        idx = dst_v[pl.ds(i * 16, 16)]
        plsc.addupdate_scatter(hist_v, [idx], ones)
        return 0
    lax.fori_loop(0, EPW // 16, _acc, 0, unroll=8)
    pltpu.sync_copy(hist_v, hist_hbm.at[pl.ds(w * N, N)])


_deg_call = pl.kernel(
    _deg_body,
    out_type=jax.ShapeDtypeStruct((NW * N,), jnp.float32),
    mesh=_MESH,
    compiler_params=_SC_PARAMS,
    scratch_types=[
        pltpu.VMEM((EPW,), jnp.int32),
        pltpu.VMEM((N,), jnp.float32),
    ],
)


def _seg_body(y_hbm, src_hbm, dst_hbm, out_hbm,
              src_v, dst_v, rows0, rows1, zbuf, acc, sem0, sem1):
    w = _wid()
    sid = lax.axis_index("s")
    cid = lax.axis_index("c")
    pltpu.sync_copy(src_hbm.at[pl.ds(w * EPW, EPW)], src_v)
    pltpu.sync_copy(dst_hbm.at[w], dst_v)

    def _sl(j):
        return src_v.at[pl.ds(j * CHUNK, CHUNK)]

    pltpu.async_copy(y_hbm.at[_sl(0)], rows0, sem0)
    pltpu.async_copy(y_hbm.at[_sl(1)], rows1, sem1)

    zeros = jnp.zeros((16,), jnp.float32)

    def _z(i, _):
        zbuf[i // 8, pl.ds((i % 8) * 16, 16)] = zeros
        return 0
    lax.fori_loop(0, 64, _z, 0, unroll=8)

    def _zc(k, _):
        pltpu.sync_copy(zbuf, acc.at[pl.ds(sid * RPS + k * 8, 8)])
        return 0
    lax.fori_loop(0, RPS // 8, _zc, 0)
    plsc.subcore_barrier()

    def _step(j, _):
        even = j % 2 == 0

        @pl.when(even)
        def _():
            pltpu.make_async_copy(y_hbm.at[_sl(j)], rows0, sem0).wait()
            pltpu.sync_copy(rows0, acc.at[dst_v.at[j]], add=True)

            @pl.when(j + 2 < NCHUNK)
            def _():
                pltpu.async_copy(y_hbm.at[_sl(j + 2)], rows0, sem0)

        @pl.when(jnp.logical_not(even))
        def _():
            pltpu.make_async_copy(y_hbm.at[_sl(j)], rows1, sem1).wait()
            pltpu.sync_copy(rows1, acc.at[dst_v.at[j]], add=True)

            @pl.when(j + 2 < NCHUNK)
            def _():
                pltpu.async_copy(y_hbm.at[_sl(j + 2)], rows1, sem1)
        return 0

    lax.fori_loop(0, NCHUNK, _step, 0)
    plsc.subcore_barrier()
    pltpu.sync_copy(acc.at[pl.ds(sid * RPS, RPS)],
                    out_hbm.at[pl.ds(cid * N16 + sid * RPS, RPS)])


_seg_call = pl.kernel(
    _seg_body,
    out_type=jax.ShapeDtypeStruct((NC * N16, D), jnp.float32),
    mesh=_MESH,
    compiler_params=_SC_PARAMS,
    scratch_types=[
        pltpu.VMEM((EPW,), jnp.int32),
        pltpu.VMEM((NCHUNK, CHUNK), jnp.int32),
        pltpu.VMEM((CHUNK, D), jnp.float32),
        pltpu.VMEM((CHUNK, D), jnp.float32),
        pltpu.VMEM((8, D), jnp.float32),
        pltpu.VMEM_SHARED((N16, D), jnp.float32),
        pltpu.SemaphoreType.DMA,
        pltpu.SemaphoreType.DMA,
    ],
)


def _make_edge(ne, ck):
    epw = ne // NW
    nchunk = epw // ck

    def _edge_body(a_hbm, b_hbm, src_hbm, dst_hbm, z_hbm,
                   src_v, dst_v, a0, a1, b0, b1, o0, o1,
                   sa0, sa1, sb0, sb1, so0, so1):
        w = _wid()
        base = w * epw
        pltpu.sync_copy(src_hbm.at[pl.ds(base, epw)], src_v)
        pltpu.sync_copy(dst_hbm.at[pl.ds(base, epw)], dst_v)

        def _sl(v, j):
            return v.at[pl.ds(j * ck, ck)]

        def _zsl(j):
            return z_hbm.at[pl.ds(base + j * ck, ck)]

        pltpu.async_copy(a_hbm.at[_sl(src_v, 0)], a0, sa0)
        pltpu.async_copy(b_hbm.at[_sl(dst_v, 0)], b0, sb0)

        def _wait_store(obuf, osem):
            pltpu.make_async_copy(obuf, z_hbm.at[pl.ds(base, ck)],
                                  osem).wait()

        def _proc(j, abuf, bbuf, obuf, asem, bsem, osem):
            @pl.when(j >= 2)
            def _():
                _wait_store(obuf, osem)
            pltpu.make_async_copy(a_hbm.at[_sl(src_v, j)], abuf, asem).wait()
            pltpu.make_async_copy(b_hbm.at[_sl(dst_v, j)], bbuf, bsem).wait()

            @plsc.parallel_loop(0, ck, step=1, unroll=8)
            def _c(r):
                for c in range(8):
                    col = pl.ds(c * 16, 16)
                    obuf[r, col] = abuf[r, col] + bbuf[r, col]
            pltpu.async_copy(obuf, _zsl(j), osem)

        def _step(j, _):
            even = j % 2 == 0

            @pl.when(jnp.logical_and(even, j + 1 < nchunk))
            def _():
                pltpu.async_copy(a_hbm.at[_sl(src_v, j + 1)], a1, sa1)
                pltpu.async_copy(b_hbm.at[_sl(dst_v, j + 1)], b1, sb1)

            @pl.when(jnp.logical_and(jnp.logical_not(even), j + 1 < nchunk))
            def _():
                pltpu.async_copy(a_hbm.at[_sl(src_v, j + 1)], a0, sa0)
                pltpu.async_copy(b_hbm.at[_sl(dst_v, j + 1)], b0, sb0)

            @pl.when(even)
            def _():
                _proc(j, a0, b0, o0, sa0, sb0, so0)

            @pl.when(jnp.logical_not(even))
            def _():
                _proc(j, a1, b1, o1, sa1, sb1, so1)
            return 0

        lax.fori_loop(0, nchunk, _step, 0)
        _wait_store(o0, so0)
        _wait_store(o1, so1)

    return pl.kernel(
        _edge_body,
        out_type=jax.ShapeDtypeStruct((ne, D), jnp.float32),
        mesh=_MESH,
        compiler_params=_SC_PARAMS,
        scratch_types=[
            pltpu.VMEM((epw,), jnp.int32),
            pltpu.VMEM((epw,), jnp.int32),
            pltpu.VMEM((ck, D), jnp.float32),
            pltpu.VMEM((ck, D), jnp.float32),
            pltpu.VMEM((ck, D), jnp.float32),
            pltpu.VMEM((ck, D), jnp.float32),
            pltpu.VMEM((ck, D), jnp.float32),
            pltpu.VMEM((ck, D), jnp.float32),
            pltpu.SemaphoreType.DMA,
            pltpu.SemaphoreType.DMA,
            pltpu.SemaphoreType.DMA,
            pltpu.SemaphoreType.DMA,
            pltpu.SemaphoreType.DMA,
            pltpu.SemaphoreType.DMA,
        ],
    )


_EA = 192000
_EB = E - _EA
_edge_call_a = _make_edge(_EA, CHUNK)
_edge_call_b = _make_edge(_EB, CHUNK)


_BN = 512
_BE = 3200


def _diag(v_row):
    ri = lax.broadcasted_iota(jnp.int32, (_BN, _BN), 0)
    ci = lax.broadcasted_iota(jnp.int32, (_BN, _BN), 1)
    eye = jnp.where(ri == ci, 1.0, 0.0).astype(jnp.float32)
    return eye * v_row


def _tc1_body(hist_ref, x_ref, w1_ref, y1_ref, disf_ref):
    deg = jnp.sum(hist_ref[...], axis=0, keepdims=True) + 1.0
    disd = _diag(lax.rsqrt(deg))
    xw = jnp.dot(x_ref[...], w1_ref[...], preferred_element_type=jnp.float32)
    disf = jnp.dot(disd, jnp.ones((_BN, D), jnp.float32),
                   preferred_element_type=jnp.float32)
    y1_ref[...] = xw * disf
    disf_ref[...] = disf


def _tc2_body(sa_ref, sb_ref, y1_ref, dis_ref, b1_ref, w2_ref, y2_ref):
    dis = dis_ref[...]
    h1 = jnp.maximum(
        dis * (sa_ref[...] + sb_ref[...] + y1_ref[...]) + b1_ref[...], 0.0)
    y2_ref[...] = jnp.dot(h1, w2_ref[...],
                          preferred_element_type=jnp.float32) * dis


def _tc3_body(sa_ref, sb_ref, y2_ref, dis_ref, b2_ref, la_ref, lb_ref,
              a_ref, bm_ref):
    h2 = (dis_ref[...] * (sa_ref[...] + sb_ref[...] + y2_ref[...])
          + b2_ref[...])
    a_ref[...] = jnp.dot(h2, la_ref[...], preferred_element_type=jnp.float32)
    bm_ref[...] = jnp.dot(h2, lb_ref[...], preferred_element_type=jnp.float32)


def _tc4_body(g_ref, attrt_ref, lc_ref, lb1_ref, w_ref, b_ref, ot_ref):
    p = lax.dot_general(attrt_ref[...], lc_ref[...],
                        (((0,), (0,)), ((), ())),
                        preferred_element_type=jnp.float32)
    e = jnp.maximum(g_ref[...] + p + lb1_ref[...], 0.0)
    ot_ref[...] = lax.dot_general(
        w_ref[...], e, (((0,), (1,)), ((), ())),
        preferred_element_type=jnp.float32) + b_ref[...]


def _node_spec():
    return pl.BlockSpec((_BN, D), lambda i: (i, 0))


def _full(shape):
    return pl.BlockSpec(shape, lambda i: tuple(0 for _ in shape))


def kernel(x, edge_index, edge_attr, W1, b1, W2, b2, lW1, lb1, lW2, lb2):
    f32 = jnp.float32
    src = edge_index[0]
    dst = edge_index[1]
    dst2 = dst.reshape(NW, NCHUNK, CHUNK)

    hist = _deg_call(dst)
    hist2 = hist.reshape(NW, N)

    grid_n = pl.cdiv(N, _BN)
    y1, disf = pl.pallas_call(
        _tc1_body,
        grid=(grid_n,),
        in_specs=[pl.BlockSpec((NW, _BN), lambda i: (0, i)),
                  _node_spec(), _full((D, H))],
        out_specs=[_node_spec(), _node_spec()],
        out_shape=[jax.ShapeDtypeStruct((N, H), f32),
                   jax.ShapeDtypeStruct((N, H), f32)],
    )(hist2, x, W1)

    s1 = _seg_call(y1, src, dst2)
    _sa = pl.BlockSpec((_BN, D), lambda i: (i, 0))
    _sb = pl.BlockSpec((_BN, D), lambda i: (i + N16 // _BN, 0))

    y2 = pl.pallas_call(
        _tc2_body,
        grid=(grid_n,),
        in_specs=[_sa, _sb, _node_spec(), _node_spec(),
                  _full((1, H)), _full((H, H))],
        out_specs=_node_spec(),
        out_shape=jax.ShapeDtypeStruct((N, H), f32),
    )(s1, s1, y1, disf, b1.reshape(1, H), W2)

    s2 = _seg_call(y2, src, dst2)

    a_n, b_n = pl.pallas_call(
        _tc3_body,
        grid=(grid_n,),
        in_specs=[_sa, _sb, _node_spec(), _node_spec(),
                  _full((1, H)), _full((H, H)), _full((H, H))],
        out_specs=[_node_spec(), _node_spec()],
        out_shape=[jax.ShapeDtypeStruct((N, H), f32),
                   jax.ShapeDtypeStruct((N, H), f32)],
    )(s2, s2, y2, disf, b2.reshape(1, H), lW1[:H], lW1[H:2 * H])

    attr_t = edge_attr.T
    lc = lW1[2 * H:]
    lb1r = lb1.reshape(1, H)
    lb2r = lb2.reshape(C, 1)

    def _tc4(zh, ath, ne):
        return pl.pallas_call(
            _tc4_body,
            grid=(ne // _BE,),
            in_specs=[pl.BlockSpec((_BE, H), lambda i: (i, 0)),
                      pl.BlockSpec((DE, _BE), lambda i: (0, i)),
                      _full((DE, H)), _full((1, H)), _full((H, C)),
                      _full((C, 1))],
            out_specs=pl.BlockSpec((C, _BE), lambda i: (0, i)),
            out_shape=jax.ShapeDtypeStruct((C, ne), f32),
        )(zh, ath, lc, lb1r, lW2, lb2r)

    z_a = _edge_call_a(a_n, b_n, src[:_EA], dst[:_EA])
    z_b = _edge_call_b(a_n, b_n, src[_EA:], dst[_EA:])
    out_a = _tc4(z_a, attr_t[:, :_EA], _EA)
    out_b = _tc4(z_b, attr_t[:, _EA:], _EB)
    return jnp.concatenate([out_a, out_b], axis=1).T


# speedup vs baseline: 18.0281x; 1.0347x over previous
import jax
import jax.numpy as jnp
from jax import lax
from jax.experimental import pallas as pl
from jax.experimental.pallas import tpu as pltpu
from jax.experimental.pallas import tpu_sc as plsc

N = 10000
E = 320000
D = 128
H = 128
DE = 16
C = 2

NC = 2
NS = 16
NW = NC * NS
EPW = E // NW
CHUNK = 80
NCHUNK = EPW // CHUNK
N16 = 10240
RPS = N16 // NS

_MESH = plsc.VectorSubcoreMesh(
    core_axis_name="c", subcore_axis_name="s", num_cores=NC, num_subcores=NS)
_SC_PARAMS = pltpu.CompilerParams(needs_layout_passes=False)


def _wid():
    return lax.axis_index("s") * NC + lax.axis_index("c")


def _deg_body(dst_hbm, hist_hbm, dst_v, hist_v):
    w = _wid()
    pltpu.sync_copy(dst_hbm.at[pl.ds(w * EPW, EPW)], dst_v)
    zeros = jnp.zeros((16,), jnp.float32)

    def _z(i, _):
        hist_v[pl.ds(i * 16, 16)] = zeros
        return 0
    lax.fori_loop(0, N // 16, _z, 0, unroll=8)

    ones = jnp.full((16,), 1.0, jnp.float32)

    def _acc(i, _):
        idx = dst_v[pl.ds(i * 16, 16)]
        plsc.addupdate_scatter(hist_v, [idx], ones)
        return 0
    lax.fori_loop(0, EPW // 16, _acc, 0, unroll=8)
    pltpu.sync_copy(hist_v, hist_hbm.at[pl.ds(w * N, N)])


_deg_call = pl.kernel(
    _deg_body,
    out_type=jax.ShapeDtypeStruct((NW * N,), jnp.float32),
    mesh=_MESH,
    compiler_params=_SC_PARAMS,
    scratch_types=[
        pltpu.VMEM((EPW,), jnp.int32),
        pltpu.VMEM((N,), jnp.float32),
    ],
)


def _seg_body(y_hbm, src_hbm, dst_hbm, out_hbm,
              src_v, dst_v, rows0, rows1, zbuf, acc, sem0, sem1):
    w = _wid()
    sid = lax.axis_index("s")
    cid = lax.axis_index("c")
    pltpu.sync_copy(src_hbm.at[pl.ds(w * EPW, EPW)], src_v)
    pltpu.sync_copy(dst_hbm.at[w], dst_v)

    def _sl(j):
        return src_v.at[pl.ds(j * CHUNK, CHUNK)]

    pltpu.async_copy(y_hbm.at[_sl(0)], rows0, sem0)
    pltpu.async_copy(y_hbm.at[_sl(1)], rows1, sem1)

    zeros = jnp.zeros((16,), jnp.float32)

    def _z(i, _):
        zbuf[i // 8, pl.ds((i % 8) * 16, 16)] = zeros
        return 0
    lax.fori_loop(0, 64, _z, 0, unroll=8)

    def _zc(k, _):
        pltpu.sync_copy(zbuf, acc.at[pl.ds(sid * RPS + k * 8, 8)])
        return 0
    lax.fori_loop(0, RPS // 8, _zc, 0)
    plsc.subcore_barrier()

    def _step(j, _):
        even = j % 2 == 0

        @pl.when(even)
        def _():
            pltpu.make_async_copy(y_hbm.at[_sl(j)], rows0, sem0).wait()
            pltpu.sync_copy(rows0, acc.at[dst_v.at[j]], add=True)

            @pl.when(j + 2 < NCHUNK)
            def _():
                pltpu.async_copy(y_hbm.at[_sl(j + 2)], rows0, sem0)

        @pl.when(jnp.logical_not(even))
        def _():
            pltpu.make_async_copy(y_hbm.at[_sl(j)], rows1, sem1).wait()
            pltpu.sync_copy(rows1, acc.at[dst_v.at[j]], add=True)

            @pl.when(j + 2 < NCHUNK)
            def _():
                pltpu.async_copy(y_hbm.at[_sl(j + 2)], rows1, sem1)
        return 0

    lax.fori_loop(0, NCHUNK, _step, 0)
    plsc.subcore_barrier()
    pltpu.sync_copy(acc.at[pl.ds(sid * RPS, RPS)],
                    out_hbm.at[pl.ds(cid * N16 + sid * RPS, RPS)])


_seg_call = pl.kernel(
    _seg_body,
    out_type=jax.ShapeDtypeStruct((NC * N16, D), jnp.float32),
    mesh=_MESH,
    compiler_params=_SC_PARAMS,
    scratch_types=[
        pltpu.VMEM((EPW,), jnp.int32),
        pltpu.VMEM((NCHUNK, CHUNK), jnp.int32),
        pltpu.VMEM((CHUNK, D), jnp.float32),
        pltpu.VMEM((CHUNK, D), jnp.float32),
        pltpu.VMEM((8, D), jnp.float32),
        pltpu.VMEM_SHARED((N16, D), jnp.float32),
        pltpu.SemaphoreType.DMA,
        pltpu.SemaphoreType.DMA,
    ],
)


def _make_edge(ne, ck):
    epw = ne // NW
    half = epw // 2
    nchunk = half // ck

    g2 = ne // 2

    def _edge_body(a_hbm, b_hbm, src_hbm, dst_hbm, z_hbm,
                   src_v, dst_v, a0, a1, b0, b1, c0, c1, d0, d1, o0, o1,
                   sa0, sa1, sb0, sb1, sc0, sc1, sd0, sd1, so0, so1):
        w = _wid()
        zbase = w * half
        pltpu.sync_copy(src_hbm.at[pl.ds(w * half, half)],
                        src_v.at[pl.ds(0, half)])
        pltpu.sync_copy(src_hbm.at[pl.ds(g2 + w * half, half)],
                        src_v.at[pl.ds(half, half)])
        pltpu.sync_copy(dst_hbm.at[pl.ds(w * half, half)],
                        dst_v.at[pl.ds(0, half)])
        pltpu.sync_copy(dst_hbm.at[pl.ds(g2 + w * half, half)],
                        dst_v.at[pl.ds(half, half)])

        def _sl(v, j):
            return v.at[pl.ds(j * ck, ck)]

        def _sh(v, j):
            return v.at[pl.ds(half + j * ck, ck)]

        def _zsl(j):
            return z_hbm.at[pl.ds(zbase + j * ck, ck)]

        def _gather(j, ab, bb, cb, db, sa, sb, sc, sd):
            pltpu.async_copy(a_hbm.at[_sl(src_v, j)], ab, sa)
            pltpu.async_copy(b_hbm.at[_sl(dst_v, j)], bb, sb)
            pltpu.async_copy(a_hbm.at[_sh(src_v, j)], cb, sc)
            pltpu.async_copy(b_hbm.at[_sh(dst_v, j)], db, sd)

        _gather(0, a0, b0, c0, d0, sa0, sb0, sc0, sd0)

        def _wait_store(obuf, osem):
            pltpu.make_async_copy(obuf, z_hbm.at[pl.ds(zbase, ck)],
                                  osem).wait()

        mhi = jnp.full((16,), -65536, jnp.int32)
        s16 = jnp.full((16,), 16, jnp.int32)

        def _proc(j, ab, bb, cb, db, obuf, sa, sb, sc, sd, osem):
            @pl.when(j >= 2)
            def _():
                _wait_store(obuf, osem)
            pltpu.make_async_copy(a_hbm.at[_sl(src_v, j)], ab, sa).wait()
            pltpu.make_async_copy(b_hbm.at[_sl(dst_v, j)], bb, sb).wait()
            pltpu.make_async_copy(a_hbm.at[_sh(src_v, j)], cb, sc).wait()
            pltpu.make_async_copy(b_hbm.at[_sh(dst_v, j)], db, sd).wait()

            @plsc.parallel_loop(0, ck, step=1, unroll=4)
            def _c(r):
                for c in range(8):
                    col = pl.ds(c * 16, 16)
                    lo = plsc.bitcast(ab[r, col] + bb[r, col], jnp.int32)
                    hi = plsc.bitcast(cb[r, col] + db[r, col], jnp.int32)
                    obuf[r, col] = (
                        lax.shift_right_logical(lo, s16) | (hi & mhi))
            pltpu.async_copy(obuf, _zsl(j), osem)

        def _step(j, _):
            even = j % 2 == 0

            @pl.when(jnp.logical_and(even, j + 1 < nchunk))
            def _():
                _gather(j + 1, a1, b1, c1, d1, sa1, sb1, sc1, sd1)

            @pl.when(jnp.logical_and(jnp.logical_not(even), j + 1 < nchunk))
            def _():
                _gather(j + 1, a0, b0, c0, d0, sa0, sb0, sc0, sd0)

            @pl.when(even)
            def _():
                _proc(j, a0, b0, c0, d0, o0, sa0, sb0, sc0, sd0, so0)

            @pl.when(jnp.logical_not(even))
            def _():
                _proc(j, a1, b1, c1, d1, o1, sa1, sb1, sc1, sd1, so1)
            return 0

        lax.fori_loop(0, nchunk, _step, 0)
        _wait_store(o0, so0)
        _wait_store(o1, so1)

    bufs = [pltpu.VMEM((ck, D), jnp.float32) for _ in range(8)]
    obufs = [pltpu.VMEM((ck, D), jnp.int32) for _ in range(2)]
    sems = [pltpu.SemaphoreType.DMA for _ in range(10)]
    return pl.kernel(
        _edge_body,
        out_type=jax.ShapeDtypeStruct((ne // 2, D), jnp.int32),
        mesh=_MESH,
        compiler_params=_SC_PARAMS,
        scratch_types=[
            pltpu.VMEM((epw,), jnp.int32),
            pltpu.VMEM((epw,), jnp.int32),
        ] + bufs + obufs + sems,
    )


_EA = 192000
_EB = E - _EA
_edge_call_a = _make_edge(_EA, 40)
_edge_call_b = _make_edge(_EB, 40)


_BN = 512
_BE = 3200


def _diag(v_row):
    ri = lax.broadcasted_iota(jnp.int32, (_BN, _BN), 0)
    ci = lax.broadcasted_iota(jnp.int32, (_BN, _BN), 1)
    eye = jnp.where(ri == ci, 1.0, 0.0).astype(jnp.float32)
    return eye * v_row


def _tc1_body(hist_ref, x_ref, w1_ref, y1_ref, disf_ref):
    deg = jnp.sum(hist_ref[...], axis=0, keepdims=True) + 1.0
    disd = _diag(lax.rsqrt(deg))
    xw = jnp.dot(x_ref[...], w1_ref[...], preferred_element_type=jnp.float32)
    disf = jnp.dot(disd, jnp.ones((_BN, D), jnp.float32),
                   preferred_element_type=jnp.float32)
    y1_ref[...] = xw * disf
    disf_ref[...] = disf


def _tc2_body(sa_ref, sb_ref, y1_ref, dis_ref, b1_ref, w2_ref, y2_ref):
    dis = dis_ref[...]
    h1 = jnp.maximum(
        dis * (sa_ref[...] + sb_ref[...] + y1_ref[...]) + b1_ref[...], 0.0)
    y2_ref[...] = jnp.dot(h1, w2_ref[...],
                          preferred_element_type=jnp.float32) * dis


def _tc3_body(sa_ref, sb_ref, y2_ref, dis_ref, b2_ref, la_ref, lb_ref,
              a_ref, bm_ref):
    h2 = (dis_ref[...] * (sa_ref[...] + sb_ref[...] + y2_ref[...])
          + b2_ref[...])
    a_ref[...] = jnp.dot(h2, la_ref[...], preferred_element_type=jnp.float32)
    bm_ref[...] = jnp.dot(h2, lb_ref[...], preferred_element_type=jnp.float32)


_BEP = 1280


def _tc4_body(g_ref, atl_ref, ath_ref, lc_ref, lb1_ref, w_ref, b_ref,
              olo_ref, ohi_ref):
    gi = g_ref[...]
    glo = lax.bitcast_convert_type(gi << 16, jnp.float32)
    ghi = lax.bitcast_convert_type(gi & jnp.int32(-65536), jnp.float32)
    lb1 = lb1_ref[...]
    w = w_ref[...]
    b = b_ref[...]

    def _half(gf, at_ref):
        p = lax.dot_general(at_ref[...], lc_ref[...],
                            (((0,), (0,)), ((), ())),
                            preferred_element_type=jnp.float32)
        e = jnp.maximum(gf + p + lb1, 0.0)
        return lax.dot_general(
            w, e, (((0,), (1,)), ((), ())),
            preferred_element_type=jnp.float32) + b

    olo_ref[...] = _half(glo, atl_ref)
    ohi_ref[...] = _half(ghi, ath_ref)


def _node_spec():
    return pl.BlockSpec((_BN, D), lambda i: (i, 0))


def _full(shape):
    return pl.BlockSpec(shape, lambda i: tuple(0 for _ in shape))


def kernel(x, edge_index, edge_attr, W1, b1, W2, b2, lW1, lb1, lW2, lb2):
    f32 = jnp.float32
    src = edge_index[0]
    dst = edge_index[1]
    dst2 = dst.reshape(NW, NCHUNK, CHUNK)

    hist = _deg_call(dst)
    hist2 = hist.reshape(NW, N)

    grid_n = pl.cdiv(N, _BN)
    y1, disf = pl.pallas_call(
        _tc1_body,
        grid=(grid_n,),
        in_specs=[pl.BlockSpec((NW, _BN), lambda i: (0, i)),
                  _node_spec(), _full((D, H))],
        out_specs=[_node_spec(), _node_spec()],
        out_shape=[jax.ShapeDtypeStruct((N, H), f32),
                   jax.ShapeDtypeStruct((N, H), f32)],
    )(hist2, x, W1)

    s1 = _seg_call(y1, src, dst2)
    _sa = pl.BlockSpec((_BN, D), lambda i: (i, 0))
    _sb = pl.BlockSpec((_BN, D), lambda i: (i + N16 // _BN, 0))

    y2 = pl.pallas_call(
        _tc2_body,
        grid=(grid_n,),
        in_specs=[_sa, _sb, _node_spec(), _node_spec(),
                  _full((1, H)), _full((H, H))],
        out_specs=_node_spec(),
        out_shape=jax.ShapeDtypeStruct((N, H), f32),
    )(s1, s1, y1, disf, b1.reshape(1, H), W2)

    s2 = _seg_call(y2, src, dst2)

    a_n, b_n = pl.pallas_call(
        _tc3_body,
        grid=(grid_n,),
        in_specs=[_sa, _sb, _node_spec(), _node_spec(),
                  _full((1, H)), _full((H, H)), _full((H, H))],
        out_specs=[_node_spec(), _node_spec()],
        out_shape=[jax.ShapeDtypeStruct((N, H), f32),
                   jax.ShapeDtypeStruct((N, H), f32)],
    )(s2, s2, y2, disf, b2.reshape(1, H), lW1[:H], lW1[H:2 * H])

    attr_t = edge_attr.T
    lc = lW1[2 * H:]
    lb1r = lb1.reshape(1, H)
    lb2r = lb2.reshape(C, 1)

    def _tc4(zh, ath, ne):
        hb = ne // 2 // _BEP

        def _hi_map(i):
            return (0, i + hb)

        return pl.pallas_call(
            _tc4_body,
            grid=(ne // 2 // _BEP,),
            in_specs=[pl.BlockSpec((_BEP, H), lambda i: (i, 0)),
                      pl.BlockSpec((DE, _BEP), lambda i: (0, i)),
                      pl.BlockSpec((DE, _BEP), _hi_map),
                      _full((DE, H)), _full((1, H)), _full((H, C)),
                      _full((C, 1))],
            out_specs=[pl.BlockSpec((C, _BEP), lambda i: (0, i)),
                       pl.BlockSpec((C, _BEP), lambda i: (0, i))],
            out_shape=[jax.ShapeDtypeStruct((C, ne // 2), f32),
                       jax.ShapeDtypeStruct((C, ne // 2), f32)],
        )(zh, ath, ath, lc, lb1r, lW2, lb2r)

    z_a = _edge_call_a(a_n, b_n, src[:_EA], dst[:_EA])
    z_b = _edge_call_b(a_n, b_n, src[_EA:], dst[_EA:])
    olo_a, ohi_a = _tc4(z_a, attr_t[:, :_EA], _EA)
    olo_b, ohi_b = _tc4(z_b, attr_t[:, _EA:], _EB)
    return jnp.concatenate([olo_a, ohi_a, olo_b, ohi_b], axis=1).T


# speedup vs baseline: 18.1080x; 1.0044x over previous
import jax
import jax.numpy as jnp
from jax import lax
from jax.experimental import pallas as pl
from jax.experimental.pallas import tpu as pltpu
from jax.experimental.pallas import tpu_sc as plsc

N = 10000
E = 320000
D = 128
H = 128
DE = 16
C = 2

NC = 2
NS = 16
NW = NC * NS
EPW = E // NW
CHUNK = 80
NCHUNK = EPW // CHUNK
N16 = 10240
RPS = N16 // NS

_MESH = plsc.VectorSubcoreMesh(
    core_axis_name="c", subcore_axis_name="s", num_cores=NC, num_subcores=NS)
_SC_PARAMS = pltpu.CompilerParams(needs_layout_passes=False)


def _wid():
    return lax.axis_index("s") * NC + lax.axis_index("c")


def _deg_body(dst_hbm, hist_hbm, dst_v, hist_v):
    w = _wid()
    pltpu.sync_copy(dst_hbm.at[pl.ds(w * EPW, EPW)], dst_v)
    zeros = jnp.zeros((16,), jnp.float32)

    def _z(i, _):
        hist_v[pl.ds(i * 16, 16)] = zeros
        return 0
    lax.fori_loop(0, N // 16, _z, 0, unroll=8)

    ones = jnp.full((16,), 1.0, jnp.float32)

    def _acc(i, _):
        idx = dst_v[pl.ds(i * 16, 16)]
        plsc.addupdate_scatter(hist_v, [idx], ones)
        return 0
    lax.fori_loop(0, EPW // 16, _acc, 0, unroll=8)
    pltpu.sync_copy(hist_v, hist_hbm.at[pl.ds(w * N, N)])


_deg_call = pl.kernel(
    _deg_body,
    out_type=jax.ShapeDtypeStruct((NW * N,), jnp.float32),
    mesh=_MESH,
    compiler_params=_SC_PARAMS,
    scratch_types=[
        pltpu.VMEM((EPW,), jnp.int32),
        pltpu.VMEM((N,), jnp.float32),
    ],
)


def _seg_body(y_hbm, src_hbm, dst_hbm, out_hbm,
              src_v, dst_v, rows0, rows1, zbuf, acc, sem0, sem1):
    w = _wid()
    sid = lax.axis_index("s")
    cid = lax.axis_index("c")
    pltpu.sync_copy(src_hbm.at[pl.ds(w * EPW, EPW)], src_v)
    pltpu.sync_copy(dst_hbm.at[w], dst_v)

    def _sl(j):
        return src_v.at[pl.ds(j * CHUNK, CHUNK)]

    pltpu.async_copy(y_hbm.at[_sl(0)], rows0, sem0)
    pltpu.async_copy(y_hbm.at[_sl(1)], rows1, sem1)

    zeros = jnp.zeros((16,), jnp.float32)

    def _z(i, _):
        zbuf[i // 8, pl.ds((i % 8) * 16, 16)] = zeros
        return 0
    lax.fori_loop(0, 64, _z, 0, unroll=8)

    def _zc(k, _):
        pltpu.sync_copy(zbuf, acc.at[pl.ds(sid * RPS + k * 8, 8)])
        return 0
    lax.fori_loop(0, RPS // 8, _zc, 0)
    plsc.subcore_barrier()

    def _step(j, _):
        even = j % 2 == 0

        @pl.when(even)
        def _():
            pltpu.make_async_copy(y_hbm.at[_sl(j)], rows0, sem0).wait()
            pltpu.sync_copy(rows0, acc.at[dst_v.at[j]], add=True)

            @pl.when(j + 2 < NCHUNK)
            def _():
                pltpu.async_copy(y_hbm.at[_sl(j + 2)], rows0, sem0)

        @pl.when(jnp.logical_not(even))
        def _():
            pltpu.make_async_copy(y_hbm.at[_sl(j)], rows1, sem1).wait()
            pltpu.sync_copy(rows1, acc.at[dst_v.at[j]], add=True)

            @pl.when(j + 2 < NCHUNK)
            def _():
                pltpu.async_copy(y_hbm.at[_sl(j + 2)], rows1, sem1)
        return 0

    lax.fori_loop(0, NCHUNK, _step, 0)
    plsc.subcore_barrier()
    pltpu.sync_copy(acc.at[pl.ds(sid * RPS, RPS)],
                    out_hbm.at[pl.ds(cid * N16 + sid * RPS, RPS)])


_seg_call = pl.kernel(
    _seg_body,
    out_type=jax.ShapeDtypeStruct((NC * N16, D), jnp.float32),
    mesh=_MESH,
    compiler_params=_SC_PARAMS,
    scratch_types=[
        pltpu.VMEM((EPW,), jnp.int32),
        pltpu.VMEM((NCHUNK, CHUNK), jnp.int32),
        pltpu.VMEM((CHUNK, D), jnp.float32),
        pltpu.VMEM((CHUNK, D), jnp.float32),
        pltpu.VMEM((8, D), jnp.float32),
        pltpu.VMEM_SHARED((N16, D), jnp.float32),
        pltpu.SemaphoreType.DMA,
        pltpu.SemaphoreType.DMA,
    ],
)


def _make_edge(ne, ck):
    epw = ne // NW
    half = epw // 2
    nchunk = half // ck

    g2 = ne // 2

    def _edge_body(a_hbm, b_hbm, src_hbm, dst_hbm, z_hbm,
                   src_v, dst_v, a0, a1, b0, b1, c0, c1, d0, d1, o0, o1,
                   sa0, sa1, sb0, sb1, sc0, sc1, sd0, sd1, so0, so1):
        w = _wid()
        zbase = w * half
        pltpu.sync_copy(src_hbm.at[pl.ds(w * half, half)],
                        src_v.at[pl.ds(0, half)])
        pltpu.sync_copy(src_hbm.at[pl.ds(g2 + w * half, half)],
                        src_v.at[pl.ds(half, half)])
        pltpu.sync_copy(dst_hbm.at[pl.ds(w * half, half)],
                        dst_v.at[pl.ds(0, half)])
        pltpu.sync_copy(dst_hbm.at[pl.ds(g2 + w * half, half)],
                        dst_v.at[pl.ds(half, half)])

        def _sl(v, j):
            return v.at[pl.ds(j * ck, ck)]

        def _sh(v, j):
            return v.at[pl.ds(half + j * ck, ck)]

        def _zsl(j):
            return z_hbm.at[pl.ds(zbase + j * ck, ck)]

        def _gather(j, ab, bb, cb, db, sa, sb, sc, sd):
            pltpu.async_copy(a_hbm.at[_sl(src_v, j)], ab, sa)
            pltpu.async_copy(b_hbm.at[_sl(dst_v, j)], bb, sb)
            pltpu.async_copy(a_hbm.at[_sh(src_v, j)], cb, sc)
            pltpu.async_copy(b_hbm.at[_sh(dst_v, j)], db, sd)

        _gather(0, a0, b0, c0, d0, sa0, sb0, sc0, sd0)

        def _wait_store(obuf, osem):
            pltpu.make_async_copy(obuf, z_hbm.at[pl.ds(zbase, ck)],
                                  osem).wait()

        mhi = jnp.full((16,), -65536, jnp.int32)
        s16 = jnp.full((16,), 16, jnp.int32)

        def _proc(j, ab, bb, cb, db, obuf, sa, sb, sc, sd, osem):
            @pl.when(j >= 2)
            def _():
                _wait_store(obuf, osem)
            pltpu.make_async_copy(a_hbm.at[_sl(src_v, j)], ab, sa).wait()
            pltpu.make_async_copy(b_hbm.at[_sl(dst_v, j)], bb, sb).wait()
            pltpu.make_async_copy(a_hbm.at[_sh(src_v, j)], cb, sc).wait()
            pltpu.make_async_copy(b_hbm.at[_sh(dst_v, j)], db, sd).wait()

            @plsc.parallel_loop(0, ck, step=1, unroll=4)
            def _c(r):
                for c in range(8):
                    col = pl.ds(c * 16, 16)
                    lo = plsc.bitcast(ab[r, col] + bb[r, col], jnp.int32)
                    hi = plsc.bitcast(cb[r, col] + db[r, col], jnp.int32)
                    obuf[r, col] = (
                        lax.shift_right_logical(lo, s16) | (hi & mhi))
            pltpu.async_copy(obuf, _zsl(j), osem)

        def _step(j, _):
            even = j % 2 == 0

            @pl.when(jnp.logical_and(even, j + 1 < nchunk))
            def _():
                _gather(j + 1, a1, b1, c1, d1, sa1, sb1, sc1, sd1)

            @pl.when(jnp.logical_and(jnp.logical_not(even), j + 1 < nchunk))
            def _():
                _gather(j + 1, a0, b0, c0, d0, sa0, sb0, sc0, sd0)

            @pl.when(even)
            def _():
                _proc(j, a0, b0, c0, d0, o0, sa0, sb0, sc0, sd0, so0)

            @pl.when(jnp.logical_not(even))
            def _():
                _proc(j, a1, b1, c1, d1, o1, sa1, sb1, sc1, sd1, so1)
            return 0

        lax.fori_loop(0, nchunk, _step, 0)
        _wait_store(o0, so0)
        _wait_store(o1, so1)

    bufs = [pltpu.VMEM((ck, D), jnp.float32) for _ in range(8)]
    obufs = [pltpu.VMEM((ck, D), jnp.int32) for _ in range(2)]
    sems = [pltpu.SemaphoreType.DMA for _ in range(10)]
    return pl.kernel(
        _edge_body,
        out_type=jax.ShapeDtypeStruct((ne // 2, D), jnp.int32),
        mesh=_MESH,
        compiler_params=_SC_PARAMS,
        scratch_types=[
            pltpu.VMEM((epw,), jnp.int32),
            pltpu.VMEM((epw,), jnp.int32),
        ] + bufs + obufs + sems,
    )


_EA = 192000
_EB = E - _EA
_edge_call_a = _make_edge(_EA, 40)
_edge_call_b = _make_edge(_EB, 80)


_BN = 512
_BE = 3200


def _diag(v_row):
    ri = lax.broadcasted_iota(jnp.int32, (_BN, _BN), 0)
    ci = lax.broadcasted_iota(jnp.int32, (_BN, _BN), 1)
    eye = jnp.where(ri == ci, 1.0, 0.0).astype(jnp.float32)
    return eye * v_row


def _tc1_body(hist_ref, x_ref, w1_ref, y1_ref, disf_ref):
    deg = jnp.sum(hist_ref[...], axis=0, keepdims=True) + 1.0
    disd = _diag(lax.rsqrt(deg))
    xw = jnp.dot(x_ref[...], w1_ref[...], preferred_element_type=jnp.float32)
    disf = jnp.dot(disd, jnp.ones((_BN, D), jnp.float32),
                   preferred_element_type=jnp.float32)
    y1_ref[...] = xw * disf
    disf_ref[...] = disf


def _tc2_body(sa_ref, sb_ref, y1_ref, dis_ref, b1_ref, w2_ref, y2_ref):
    dis = dis_ref[...]
    h1 = jnp.maximum(
        dis * (sa_ref[...] + sb_ref[...] + y1_ref[...]) + b1_ref[...], 0.0)
    y2_ref[...] = jnp.dot(h1, w2_ref[...],
                          preferred_element_type=jnp.float32) * dis


def _tc3_body(sa_ref, sb_ref, y2_ref, dis_ref, b2_ref, la_ref, lb_ref,
              a_ref, bm_ref):
    h2 = (dis_ref[...] * (sa_ref[...] + sb_ref[...] + y2_ref[...])
          + b2_ref[...])
    a_ref[...] = jnp.dot(h2, la_ref[...], preferred_element_type=jnp.float32)
    bm_ref[...] = jnp.dot(h2, lb_ref[...], preferred_element_type=jnp.float32)


_BEP = 1280


def _tc4_body(g_ref, atl_ref, ath_ref, lc_ref, lb1_ref, w_ref, b_ref,
              olo_ref, ohi_ref):
    gi = g_ref[...]
    glo = lax.bitcast_convert_type(gi << 16, jnp.float32)
    ghi = lax.bitcast_convert_type(gi & jnp.int32(-65536), jnp.float32)
    lb1 = lb1_ref[...]
    w = w_ref[...]
    b = b_ref[...]

    def _half(gf, at_ref):
        p = lax.dot_general(at_ref[...], lc_ref[...],
                            (((0,), (0,)), ((), ())),
                            preferred_element_type=jnp.float32)
        e = jnp.maximum(gf + p + lb1, 0.0)
        return lax.dot_general(
            w, e, (((0,), (1,)), ((), ())),
            preferred_element_type=jnp.float32) + b

    olo_ref[...] = _half(glo, atl_ref)
    ohi_ref[...] = _half(ghi, ath_ref)


def _node_spec():
    return pl.BlockSpec((_BN, D), lambda i: (i, 0))


def _full(shape):
    return pl.BlockSpec(shape, lambda i: tuple(0 for _ in shape))


def kernel(x, edge_index, edge_attr, W1, b1, W2, b2, lW1, lb1, lW2, lb2):
    f32 = jnp.float32
    src = edge_index[0]
    dst = edge_index[1]
    dst2 = dst.reshape(NW, NCHUNK, CHUNK)

    hist = _deg_call(dst)
    hist2 = hist.reshape(NW, N)

    grid_n = pl.cdiv(N, _BN)
    y1, disf = pl.pallas_call(
        _tc1_body,
        grid=(grid_n,),
        in_specs=[pl.BlockSpec((NW, _BN), lambda i: (0, i)),
                  _node_spec(), _full((D, H))],
        out_specs=[_node_spec(), _node_spec()],
        out_shape=[jax.ShapeDtypeStruct((N, H), f32),
                   jax.ShapeDtypeStruct((N, H), f32)],
    )(hist2, x, W1)

    s1 = _seg_call(y1, src, dst2)
    _sa = pl.BlockSpec((_BN, D), lambda i: (i, 0))
    _sb = pl.BlockSpec((_BN, D), lambda i: (i + N16 // _BN, 0))

    y2 = pl.pallas_call(
        _tc2_body,
        grid=(grid_n,),
        in_specs=[_sa, _sb, _node_spec(), _node_spec(),
                  _full((1, H)), _full((H, H))],
        out_specs=_node_spec(),
        out_shape=jax.ShapeDtypeStruct((N, H), f32),
    )(s1, s1, y1, disf, b1.reshape(1, H), W2)

    s2 = _seg_call(y2, src, dst2)

    a_n, b_n = pl.pallas_call(
        _tc3_body,
        grid=(grid_n,),
        in_specs=[_sa, _sb, _node_spec(), _node_spec(),
                  _full((1, H)), _full((H, H)), _full((H, H))],
        out_specs=[_node_spec(), _node_spec()],
        out_shape=[jax.ShapeDtypeStruct((N, H), f32),
                   jax.ShapeDtypeStruct((N, H), f32)],
    )(s2, s2, y2, disf, b2.reshape(1, H), lW1[:H], lW1[H:2 * H])

    attr_t = edge_attr.T
    lc = lW1[2 * H:]
    lb1r = lb1.reshape(1, H)
    lb2r = lb2.reshape(C, 1)

    def _tc4(zh, ath, ne):
        hb = ne // 2 // _BEP

        def _hi_map(i):
            return (0, i + hb)

        return pl.pallas_call(
            _tc4_body,
            grid=(ne // 2 // _BEP,),
            in_specs=[pl.BlockSpec((_BEP, H), lambda i: (i, 0)),
                      pl.BlockSpec((DE, _BEP), lambda i: (0, i)),
                      pl.BlockSpec((DE, _BEP), _hi_map),
                      _full((DE, H)), _full((1, H)), _full((H, C)),
                      _full((C, 1))],
            out_specs=[pl.BlockSpec((C, _BEP), lambda i: (0, i)),
                       pl.BlockSpec((C, _BEP), lambda i: (0, i))],
            out_shape=[jax.ShapeDtypeStruct((C, ne // 2), f32),
                       jax.ShapeDtypeStruct((C, ne // 2), f32)],
        )(zh, ath, ath, lc, lb1r, lW2, lb2r)

    z_a = _edge_call_a(a_n, b_n, src[:_EA], dst[:_EA])
    z_b = _edge_call_b(a_n, b_n, src[_EA:], dst[_EA:])
    olo_a, ohi_a = _tc4(z_a, attr_t[:, :_EA], _EA)
    olo_b, ohi_b = _tc4(z_b, attr_t[:, _EA:], _EB)
    return jnp.concatenate([olo_a, ohi_a, olo_b, ohi_b], axis=1).T
